# baseline (speedup 1.0000x reference)
import functools

import jax
import jax.numpy as jnp
from jax import lax
from jax.experimental import pallas as pl
from jax.experimental.pallas import tpu as pltpu
from jax.experimental.pallas import tpu_sc as plsc

N = 10000
E = 320000
D = 128
NC, NS, L = 2, 16, 16
NW = NC * NS
EPW = E // NW
K = 40
NB = EPW // K
ROWS_T = 624
TAIL = N - NS * ROWS_T
BN = 1000


def _tc1_body(x_ref, w_ref, pe_ref, pr_ref, h_ref, elr_ref, rle_ref, mx_ref):
    i = pl.program_id(0)
    h = jnp.dot(x_ref[...], w_ref[...], preferred_element_type=jnp.float32)
    h_ref[...] = h
    t = jnp.dot(h, pe_ref[...], preferred_element_type=jnp.float32)
    elr_ref[...] = t
    rle_ref[...] = jnp.dot(h, pr_ref[...], preferred_element_type=jnp.float32)

    @pl.when(i == 0)
    def _():
        mx_ref[...] = jnp.full((1, L), -1e30, jnp.float32)

    mx_ref[...] = jnp.maximum(mx_ref[...], jnp.max(t, axis=0, keepdims=True))


def _tc_project(x, W, Pe, Pr):
    n, d = x.shape
    r = W.shape[1]
    return pl.pallas_call(
        _tc1_body,
        grid=(n // BN,),
        in_specs=[pl.BlockSpec((BN, d), lambda i: (i, 0)),
                  pl.BlockSpec((d, r), lambda i: (0, 0)),
                  pl.BlockSpec((r, L), lambda i: (0, 0)),
                  pl.BlockSpec((r, L), lambda i: (0, 0))],
        out_specs=[pl.BlockSpec((BN, r), lambda i: (i, 0)),
                   pl.BlockSpec((BN, L), lambda i: (i, 0)),
                   pl.BlockSpec((BN, L), lambda i: (i, 0)),
                   pl.BlockSpec((1, L), lambda i: (0, 0))],
        out_shape=[jax.ShapeDtypeStruct((n, r), jnp.float32),
                   jax.ShapeDtypeStruct((n, L), jnp.float32),
                   jax.ShapeDtypeStruct((n, L), jnp.float32),
                   jax.ShapeDtypeStruct((1, L), jnp.float32)],
    )(x, W, Pe, Pr)


def _tc2_body(p_ref, d_ref, w_ref, q_ref, b_ref, pe_ref, pr_ref,
              h2_ref, elr_ref, rle_ref, mx_ref):
    i = pl.program_id(0)
    num = p_ref[0] + p_ref[1]
    den = d_ref[0] + d_ref[1]
    den128 = jnp.dot(den, q_ref[...], preferred_element_type=jnp.float32)
    x2 = num / (den128 + 1e-9) + b_ref[...]
    x2 = jnp.maximum(x2, 0.0)
    h2 = jnp.dot(x2, w_ref[...], preferred_element_type=jnp.float32)
    h2_ref[...] = h2
    t = jnp.dot(h2, pe_ref[...], preferred_element_type=jnp.float32)
    elr_ref[...] = t
    rle_ref[...] = jnp.dot(h2, pr_ref[...], preferred_element_type=jnp.float32)

    @pl.when(i == 0)
    def _():
        mx_ref[...] = jnp.full((1, L), -1e30, jnp.float32)

    mx_ref[...] = jnp.maximum(mx_ref[...], jnp.max(t, axis=0, keepdims=True))


def _tc_combine_project(outp, denp, W2, Q, b1row, Pe, Pr):
    return pl.pallas_call(
        _tc2_body,
        grid=(N // BN,),
        in_specs=[pl.BlockSpec((NC, BN, D), lambda i: (0, i, 0)),
                  pl.BlockSpec((NC, BN, L), lambda i: (0, i, 0)),
                  pl.BlockSpec((D, L), lambda i: (0, 0)),
                  pl.BlockSpec((L, D), lambda i: (0, 0)),
                  pl.BlockSpec((1, D), lambda i: (0, 0)),
                  pl.BlockSpec((L, L), lambda i: (0, 0)),
                  pl.BlockSpec((L, L), lambda i: (0, 0))],
        out_specs=[pl.BlockSpec((BN, L), lambda i: (i, 0)),
                   pl.BlockSpec((BN, L), lambda i: (i, 0)),
                   pl.BlockSpec((BN, L), lambda i: (i, 0)),
                   pl.BlockSpec((1, L), lambda i: (0, 0))],
        out_shape=[jax.ShapeDtypeStruct((N, L), jnp.float32),
                   jax.ShapeDtypeStruct((N, L), jnp.float32),
                   jax.ShapeDtypeStruct((N, L), jnp.float32),
                   jax.ShapeDtypeStruct((1, L), jnp.float32)],
    )(outp, denp, W2, Q, b1row, Pe, Pr)


def _tc3_body(p_ref, d_ref, q2_ref, b_ref, o_ref):
    num = p_ref[0] + p_ref[1]
    den = d_ref[0] + d_ref[1]
    den16 = jnp.dot(den, q2_ref[...], preferred_element_type=jnp.float32)
    o_ref[...] = num / (den16 + 1e-9) + b_ref[...]


def _tc_finish(outp, denp, Q2, b2row):
    return pl.pallas_call(
        _tc3_body,
        grid=(N // BN,),
        in_specs=[pl.BlockSpec((NC, BN, L), lambda i: (0, i, 0)),
                  pl.BlockSpec((NC, BN, L), lambda i: (0, i, 0)),
                  pl.BlockSpec((L, L), lambda i: (0, 0)),
                  pl.BlockSpec((1, L), lambda i: (0, 0))],
        out_specs=pl.BlockSpec((BN, L), lambda i: (i, 0)),
        out_shape=jax.ShapeDtypeStruct((N, L), jnp.float32),
    )(outp, denp, Q2, b2row)


_BCAST_DNUMS = lax.GatherDimensionNumbers(
    offset_dims=(), collapsed_slice_dims=(0,), start_index_map=(0,))


def _lane_bcast(v, j):
    idx = jnp.full((L, 1), j, jnp.int32)
    return lax.gather(v, idx, _BCAST_DNUMS, (1,),
                      mode=lax.GatherScatterMode.PROMISE_IN_BOUNDS)


def _make_edge_sweep(R):
    RC = R // L
    mesh = plsc.VectorSubcoreMesh(core_axis_name="c", subcore_axis_name="s")

    slot_types = (
        pltpu.VMEM((K,), jnp.int32),
        pltpu.VMEM((K,), jnp.int32),
        pltpu.VMEM((K,), jnp.int32),
        pltpu.VMEM((K, L), jnp.float32),
        pltpu.VMEM((K, L), jnp.float32),
        pltpu.VMEM((K, L), jnp.float32),
        pltpu.VMEM((K, R), jnp.float32),
        pltpu.VMEM((K, R), jnp.float32),
        pltpu.SemaphoreType.DMA,
        pltpu.SemaphoreType.DMA,
    )

    @functools.partial(
        pl.kernel,
        out_type=(jax.ShapeDtypeStruct((NC, N, R), jnp.float32),
                  jax.ShapeDtypeStruct((NC, N, L), jnp.float32)),
        mesh=mesh,
        compiler_params=pltpu.CompilerParams(use_tc_tiling_on_sc=False),
        scratch_types=(
            pltpu.VMEM_SHARED((N, R), jnp.float32),
            pltpu.VMEM_SHARED((N, L), jnp.float32),
            pltpu.VMEM((L,), jnp.float32),
        ) + slot_types + slot_types,
    )
    def sweep(src_hbm, dst_hbm, h_hbm, elr_hbm, rle_hbm, m_hbm,
              out_hbm, den_hbm,
              out_sp, den_sp, m_v, *slot_refs):
        ns = len(slot_types)
        slots = (slot_refs[:ns], slot_refs[ns:])
        cid = lax.axis_index("c")
        tid = lax.axis_index("s")
        wid = cid * NS + tid
        rbase = tid * ROWS_T

        zmsg = slots[0][7]
        zw = slots[0][5]

        def zo(i, c):
            for j in range(RC):
                zmsg[i, pl.ds(j * L, L)] = jnp.zeros((L,), jnp.float32)
            zw[i, :] = jnp.zeros((L,), jnp.float32)
            return c

        lax.fori_loop(0, K, zo, 0, unroll=4)
        for z in range(ROWS_T // K):
            pltpu.sync_copy(zmsg, out_sp.at[pl.ds(rbase + z * K, K)])
            pltpu.sync_copy(zw, den_sp.at[pl.ds(rbase + z * K, K)])
        rem = ROWS_T - (ROWS_T // K) * K
        pltpu.sync_copy(zmsg.at[pl.ds(0, rem)],
                        out_sp.at[pl.ds(rbase + ROWS_T - rem, rem)])
        pltpu.sync_copy(zw.at[pl.ds(0, rem)],
                        den_sp.at[pl.ds(rbase + ROWS_T - rem, rem)])

        @pl.when(tid == NS - 1)
        def _():
            tb = NS * ROWS_T
            pltpu.sync_copy(zmsg.at[pl.ds(0, TAIL)],
                            out_sp.at[pl.ds(tb, TAIL)])
            pltpu.sync_copy(zw.at[pl.ds(0, TAIL)],
                            den_sp.at[pl.ds(tb, TAIL)])

        pltpu.sync_copy(m_hbm, m_v)
        plsc.subcore_barrier()

        mvec = m_v[...]
        lane = lax.iota(jnp.int32, L)
        headmask = lane < 8
        ebase = wid * EPW

        def issue_gathers(s, b):
            src_v, dst_v, _, elrs_v, rled_v, _, rows_v, _, semg, _ = slots[s]
            gb = pl.multiple_of(ebase + b * K, 8)
            pltpu.sync_copy(src_hbm.at[pl.ds(gb, K)], src_v)
            pltpu.sync_copy(dst_hbm.at[pl.ds(gb, K)], dst_v)
            pltpu.async_copy(elr_hbm.at[src_v], elrs_v, semg)
            pltpu.async_copy(rle_hbm.at[dst_v], rled_v, semg)
            pltpu.async_copy(h_hbm.at[src_v], rows_v, semg)

        def wait_gathers(s):
            src_v, dst_v, _, elrs_v, rled_v, _, rows_v, _, semg, _ = slots[s]
            pltpu.make_async_copy(elr_hbm.at[src_v], elrs_v, semg).wait()
            pltpu.make_async_copy(rle_hbm.at[dst_v], rled_v, semg).wait()
            pltpu.make_async_copy(h_hbm.at[src_v], rows_v, semg).wait()

        def wait_scatters(s):
            _, _, dsc_v, _, _, w_v, _, msg_v, _, sems = slots[s]
            pltpu.make_async_copy(w_v, den_sp.at[dsc_v], sems).wait()
            pltpu.make_async_copy(msg_v, out_sp.at[dsc_v], sems).wait()

        def compute_and_scatter(s):
            _, dst_v, dsc_v, elrs_v, rled_v, w_v, rows_v, msg_v, _, sems = slots[s]

            def edge_w(k, c2):
                e = elrs_v[k, :] + rled_v[k, :]
                e = jnp.where(e > 0, e, 0.2 * e)
                w_v[k, :] = jnp.where(headmask, jnp.exp(e - mvec), 0.0)
                return c2

            lax.fori_loop(0, K, edge_w, 0, unroll=8)

            def edge_m(k, c2):
                wrow = w_v[k, :]
                for j in range(RC):
                    wj = _lane_bcast(wrow, j)
                    msg_v[k, pl.ds(j * L, L)] = rows_v[k, pl.ds(j * L, L)] * wj
                return c2

            lax.fori_loop(0, K, edge_m, 0, unroll=(8 if RC == 1 else 2))
            for i in range(K // L):
                dsc_v[pl.ds(i * L, L)] = dst_v[pl.ds(i * L, L)]
            if K % L:
                dsc_v[pl.ds(K - L, L)] = dst_v[pl.ds(K - L, L)]
            pltpu.async_copy(w_v, den_sp.at[dsc_v], sems, add=True)
            pltpu.async_copy(msg_v, out_sp.at[dsc_v], sems, add=True)

        NPAIRS = NB // 2

        issue_gathers(0, 0)
        issue_gathers(1, 1)

        def pair(p, c):
            wait_gathers(0)

            @pl.when(p > 0)
            def _():
                wait_scatters(0)

            compute_and_scatter(0)

            @pl.when(2 * p + 2 < NB)
            def _():
                issue_gathers(0, 2 * p + 2)

            wait_gathers(1)

            @pl.when(p > 0)
            def _():
                wait_scatters(1)

            compute_and_scatter(1)

            @pl.when(2 * p + 3 < NB)
            def _():
                issue_gathers(1, 2 * p + 3)

            return c

        lax.fori_loop(0, NPAIRS, pair, 0)

        wait_scatters(0)
        wait_scatters(1)

        plsc.subcore_barrier()
        pltpu.sync_copy(out_sp.at[pl.ds(rbase, ROWS_T)],
                        out_hbm.at[cid, pl.ds(rbase, ROWS_T)])
        pltpu.sync_copy(den_sp.at[pl.ds(rbase, ROWS_T)],
                        den_hbm.at[cid, pl.ds(rbase, ROWS_T)])

        @pl.when(tid == NS - 1)
        def _():
            tb = NS * ROWS_T
            pltpu.sync_copy(out_sp.at[pl.ds(tb, TAIL)],
                            out_hbm.at[cid, pl.ds(tb, TAIL)])
            pltpu.sync_copy(den_sp.at[pl.ds(tb, TAIL)],
                            den_hbm.at[cid, pl.ds(tb, TAIL)])

    return sweep


_edge_sweep_128 = _make_edge_sweep(D)
_edge_sweep_16 = _make_edge_sweep(L)


def _pack(al, ar):
    H, Fo = al.shape
    eye = jnp.eye(H, 8, dtype=jnp.float32)
    a_el = (al[:, :, None] * eye[:, None, :]).reshape(H * Fo, 8)
    a_er = (ar[:, :, None] * eye[:, None, :]).reshape(H * Fo, 8)
    return (jnp.concatenate([a_el, a_er], 1).astype(jnp.float32),
            jnp.concatenate([a_er, a_el], 1).astype(jnp.float32))


def _mtile(mx, H):
    m = mx[0]
    s = m[:8] + m[8:]
    s = jnp.where(s > 0, s, 0.2 * s)
    head = jnp.where(jnp.arange(8) < H, s, 1e30)
    return jnp.concatenate([head, jnp.full((8,), 1e30, jnp.float32)])


def kernel(feats, edge_index, W1, al1, ar1, b1, W2, al2, ar2, b2):
    pe1, pr1 = _pack(al1, ar1)
    pe2, pr2 = _pack(al2, ar2)
    q = (jnp.eye(L, 8, dtype=jnp.float32)[:, :, None]
         * jnp.ones((1, 1, L), jnp.float32)).reshape(L, D)
    q2 = jnp.zeros((L, L), jnp.float32).at[0, :].set(1.0)

    src = edge_index[0]
    dst = edge_index[1]
    h1, elr1, rle1, mx1 = _tc_project(feats, W1, pe1, pr1)
    m1 = _mtile(mx1, 8)
    out1p, den1p = _edge_sweep_128(src, dst, h1, elr1, rle1, m1)

    h2, elr2, rle2, mx2 = _tc_combine_project(
        out1p, den1p, W2, q, b1.reshape(1, D), pe2, pr2)
    m2 = _mtile(mx2, 1)
    out2p, den2p = _edge_sweep_16(src, dst, h2, elr2, rle2, m2)

    return _tc_finish(out2p, den2p, q2, b2.reshape(1, L))


# speedup vs baseline: 1.8723x; 1.8723x over previous
import functools

import jax
import jax.numpy as jnp
from jax import lax
from jax.experimental import pallas as pl
from jax.experimental.pallas import tpu as pltpu
from jax.experimental.pallas import tpu_sc as plsc

N = 10000
E = 320000
D = 128
NC, NS, L = 2, 16, 16
NW = NC * NS
EPW = E // NW
K = 40
NB = EPW // K
ROWS_T = 624
TAIL = N - NS * ROWS_T
BN = 1000


def _tc1_body(x_ref, w_ref, pe_ref, pr_ref, h_ref, elr_ref, rle_ref, mx_ref):
    i = pl.program_id(0)
    h = jnp.dot(x_ref[...], w_ref[...], preferred_element_type=jnp.float32)
    h_ref[...] = h
    t = jnp.dot(h, pe_ref[...], preferred_element_type=jnp.float32)
    elr_ref[...] = t
    rle_ref[...] = jnp.dot(h, pr_ref[...], preferred_element_type=jnp.float32)

    @pl.when(i == 0)
    def _():
        mx_ref[...] = jnp.full((1, L), -1e30, jnp.float32)

    mx_ref[...] = jnp.maximum(mx_ref[...], jnp.max(t, axis=0, keepdims=True))


def _tc_project(x, W, Pe, Pr):
    n, d = x.shape
    r = W.shape[1]
    return pl.pallas_call(
        _tc1_body,
        grid=(n // BN,),
        in_specs=[pl.BlockSpec((BN, d), lambda i: (i, 0)),
                  pl.BlockSpec((d, r), lambda i: (0, 0)),
                  pl.BlockSpec((r, L), lambda i: (0, 0)),
                  pl.BlockSpec((r, L), lambda i: (0, 0))],
        out_specs=[pl.BlockSpec((BN, r), lambda i: (i, 0)),
                   pl.BlockSpec((BN, L), lambda i: (i, 0)),
                   pl.BlockSpec((BN, L), lambda i: (i, 0)),
                   pl.BlockSpec((1, L), lambda i: (0, 0))],
        out_shape=[jax.ShapeDtypeStruct((n, r), jnp.float32),
                   jax.ShapeDtypeStruct((n, L), jnp.float32),
                   jax.ShapeDtypeStruct((n, L), jnp.float32),
                   jax.ShapeDtypeStruct((1, L), jnp.float32)],
    )(x, W, Pe, Pr)


def _tc2_body(p_ref, d_ref, w_ref, q_ref, b_ref, pe_ref, pr_ref,
              h2_ref, elr_ref, rle_ref, mx_ref):
    i = pl.program_id(0)
    num = p_ref[0] + p_ref[1]
    den = d_ref[0] + d_ref[1]
    den128 = jnp.dot(den, q_ref[...], preferred_element_type=jnp.float32)
    x2 = num / (den128 + 1e-9) + b_ref[...]
    x2 = jnp.maximum(x2, 0.0)
    h2 = jnp.dot(x2, w_ref[...], preferred_element_type=jnp.float32)
    h2_ref[...] = h2
    t = jnp.dot(h2, pe_ref[...], preferred_element_type=jnp.float32)
    elr_ref[...] = t
    rle_ref[...] = jnp.dot(h2, pr_ref[...], preferred_element_type=jnp.float32)

    @pl.when(i == 0)
    def _():
        mx_ref[...] = jnp.full((1, L), -1e30, jnp.float32)

    mx_ref[...] = jnp.maximum(mx_ref[...], jnp.max(t, axis=0, keepdims=True))


def _tc_combine_project(outp, denp, W2, Q, b1row, Pe, Pr):
    return pl.pallas_call(
        _tc2_body,
        grid=(N // BN,),
        in_specs=[pl.BlockSpec((NC, BN, D), lambda i: (0, i, 0)),
                  pl.BlockSpec((NC, BN, L), lambda i: (0, i, 0)),
                  pl.BlockSpec((D, L), lambda i: (0, 0)),
                  pl.BlockSpec((L, D), lambda i: (0, 0)),
                  pl.BlockSpec((1, D), lambda i: (0, 0)),
                  pl.BlockSpec((L, L), lambda i: (0, 0)),
                  pl.BlockSpec((L, L), lambda i: (0, 0))],
        out_specs=[pl.BlockSpec((BN, L), lambda i: (i, 0)),
                   pl.BlockSpec((BN, L), lambda i: (i, 0)),
                   pl.BlockSpec((BN, L), lambda i: (i, 0)),
                   pl.BlockSpec((1, L), lambda i: (0, 0))],
        out_shape=[jax.ShapeDtypeStruct((N, L), jnp.float32),
                   jax.ShapeDtypeStruct((N, L), jnp.float32),
                   jax.ShapeDtypeStruct((N, L), jnp.float32),
                   jax.ShapeDtypeStruct((1, L), jnp.float32)],
    )(outp, denp, W2, Q, b1row, Pe, Pr)


def _tc3_body(p_ref, d_ref, q2_ref, b_ref, o_ref):
    num = p_ref[0] + p_ref[1]
    den = d_ref[0] + d_ref[1]
    den16 = jnp.dot(den, q2_ref[...], preferred_element_type=jnp.float32)
    o_ref[...] = num / (den16 + 1e-9) + b_ref[...]


def _tc_finish(outp, denp, Q2, b2row):
    return pl.pallas_call(
        _tc3_body,
        grid=(N // BN,),
        in_specs=[pl.BlockSpec((NC, BN, L), lambda i: (0, i, 0)),
                  pl.BlockSpec((NC, BN, L), lambda i: (0, i, 0)),
                  pl.BlockSpec((L, L), lambda i: (0, 0)),
                  pl.BlockSpec((1, L), lambda i: (0, 0))],
        out_specs=pl.BlockSpec((BN, L), lambda i: (i, 0)),
        out_shape=jax.ShapeDtypeStruct((N, L), jnp.float32),
    )(outp, denp, Q2, b2row)


_BCAST_DNUMS = lax.GatherDimensionNumbers(
    offset_dims=(), collapsed_slice_dims=(0,), start_index_map=(0,))


def _lane_bcast(v, j):
    idx = jnp.full((L, 1), j, jnp.int32)
    return lax.gather(v, idx, _BCAST_DNUMS, (1,),
                      mode=lax.GatherScatterMode.PROMISE_IN_BOUNDS)


def _make_edge_sweep(R):
    RC = R // L
    mesh = plsc.VectorSubcoreMesh(core_axis_name="c", subcore_axis_name="s")

    slot_types = (
        pltpu.VMEM((K,), jnp.int32),
        pltpu.VMEM((K,), jnp.int32),
        pltpu.VMEM((K,), jnp.int32),
        pltpu.VMEM((K, L), jnp.float32),
        pltpu.VMEM((K, L), jnp.float32),
        pltpu.VMEM((K, L), jnp.float32),
        pltpu.VMEM((K, R), jnp.float32),
        pltpu.VMEM((K, R), jnp.float32),
        pltpu.SemaphoreType.DMA,
        pltpu.SemaphoreType.DMA,
    )

    @functools.partial(
        pl.kernel,
        out_type=(jax.ShapeDtypeStruct((NC, N, R), jnp.float32),
                  jax.ShapeDtypeStruct((NC, N, L), jnp.float32)),
        mesh=mesh,
        compiler_params=pltpu.CompilerParams(use_tc_tiling_on_sc=False),
        scratch_types=(
            pltpu.VMEM_SHARED((N, R), jnp.float32),
            pltpu.VMEM_SHARED((N, L), jnp.float32),
            pltpu.VMEM((L,), jnp.float32),
        ) + slot_types + slot_types,
    )
    def sweep(src_hbm, dst_hbm, h_hbm, elr_hbm, rle_hbm, m_hbm,
              out_hbm, den_hbm,
              out_sp, den_sp, m_v, *slot_refs):
        ns = len(slot_types)
        slots = (slot_refs[:ns], slot_refs[ns:])
        cid = lax.axis_index("c")
        tid = lax.axis_index("s")
        wid = cid * NS + tid
        rbase = tid * ROWS_T

        zmsg = slots[0][7]
        zw = slots[0][5]

        def zo(i, c):
            for j in range(RC):
                zmsg[i, pl.ds(j * L, L)] = jnp.zeros((L,), jnp.float32)
            zw[i, :] = jnp.zeros((L,), jnp.float32)
            return c

        lax.fori_loop(0, K, zo, 0, unroll=4)
        for z in range(ROWS_T // K):
            pltpu.sync_copy(zmsg, out_sp.at[pl.ds(rbase + z * K, K)])
            pltpu.sync_copy(zw, den_sp.at[pl.ds(rbase + z * K, K)])
        rem = ROWS_T - (ROWS_T // K) * K
        pltpu.sync_copy(zmsg.at[pl.ds(0, rem)],
                        out_sp.at[pl.ds(rbase + ROWS_T - rem, rem)])
        pltpu.sync_copy(zw.at[pl.ds(0, rem)],
                        den_sp.at[pl.ds(rbase + ROWS_T - rem, rem)])

        @pl.when(tid == NS - 1)
        def _():
            tb = NS * ROWS_T
            pltpu.sync_copy(zmsg.at[pl.ds(0, TAIL)],
                            out_sp.at[pl.ds(tb, TAIL)])
            pltpu.sync_copy(zw.at[pl.ds(0, TAIL)],
                            den_sp.at[pl.ds(tb, TAIL)])

        pltpu.sync_copy(m_hbm, m_v)
        plsc.subcore_barrier()

        mvec = m_v[...]
        lane = lax.iota(jnp.int32, L)
        headmask = lane < 8
        ebase = wid * EPW

        def issue_gathers(s, b):
            src_v, dst_v, _, elrs_v, rled_v, _, rows_v, _, semg, _ = slots[s]
            gb = pl.multiple_of(ebase + b * K, 8)
            pltpu.sync_copy(src_hbm.at[pl.ds(gb, K)], src_v)
            pltpu.sync_copy(dst_hbm.at[pl.ds(gb, K)], dst_v)
            pltpu.async_copy(elr_hbm.at[src_v], elrs_v, semg)
            pltpu.async_copy(rle_hbm.at[dst_v], rled_v, semg)
            pltpu.async_copy(h_hbm.at[src_v], rows_v, semg)

        def wait_gathers(s):
            src_v, dst_v, _, elrs_v, rled_v, _, rows_v, _, semg, _ = slots[s]
            pltpu.make_async_copy(elr_hbm.at[src_v], elrs_v, semg).wait()
            pltpu.make_async_copy(rle_hbm.at[dst_v], rled_v, semg).wait()
            pltpu.make_async_copy(h_hbm.at[src_v], rows_v, semg).wait()

        def wait_scatters(s):
            _, _, dsc_v, _, _, w_v, _, msg_v, _, sems = slots[s]
            pltpu.make_async_copy(w_v, den_sp.at[dsc_v], sems).wait()
            pltpu.make_async_copy(msg_v, out_sp.at[dsc_v], sems).wait()

        def compute_and_scatter(s):
            _, dst_v, dsc_v, elrs_v, rled_v, w_v, rows_v, msg_v, _, sems = slots[s]

            @plsc.parallel_loop(0, K, unroll=(4 if RC == 1 else 2))
            def _(k):
                e = elrs_v[k, :] + rled_v[k, :]
                e = jnp.where(e > 0, e, 0.2 * e)
                wrow = jnp.where(headmask, jnp.exp(e - mvec), 0.0)
                w_v[k, :] = wrow
                for j in range(RC):
                    msg_v[k, pl.ds(j * L, L)] = (
                        rows_v[k, pl.ds(j * L, L)] * _lane_bcast(wrow, j))
            for i in range(K // L):
                dsc_v[pl.ds(i * L, L)] = dst_v[pl.ds(i * L, L)]
            if K % L:
                dsc_v[pl.ds(K - L, L)] = dst_v[pl.ds(K - L, L)]
            pltpu.async_copy(w_v, den_sp.at[dsc_v], sems, add=True)
            pltpu.async_copy(msg_v, out_sp.at[dsc_v], sems, add=True)

        NPAIRS = NB // 2

        issue_gathers(0, 0)
        issue_gathers(1, 1)

        def pair(p, c):
            wait_gathers(0)

            @pl.when(p > 0)
            def _():
                wait_scatters(0)

            compute_and_scatter(0)

            @pl.when(2 * p + 2 < NB)
            def _():
                issue_gathers(0, 2 * p + 2)

            wait_gathers(1)

            @pl.when(p > 0)
            def _():
                wait_scatters(1)

            compute_and_scatter(1)

            @pl.when(2 * p + 3 < NB)
            def _():
                issue_gathers(1, 2 * p + 3)

            return c

        lax.fori_loop(0, NPAIRS, pair, 0)

        wait_scatters(0)
        wait_scatters(1)

        plsc.subcore_barrier()
        pltpu.sync_copy(out_sp.at[pl.ds(rbase, ROWS_T)],
                        out_hbm.at[cid, pl.ds(rbase, ROWS_T)])
        pltpu.sync_copy(den_sp.at[pl.ds(rbase, ROWS_T)],
                        den_hbm.at[cid, pl.ds(rbase, ROWS_T)])

        @pl.when(tid == NS - 1)
        def _():
            tb = NS * ROWS_T
            pltpu.sync_copy(out_sp.at[pl.ds(tb, TAIL)],
                            out_hbm.at[cid, pl.ds(tb, TAIL)])
            pltpu.sync_copy(den_sp.at[pl.ds(tb, TAIL)],
                            den_hbm.at[cid, pl.ds(tb, TAIL)])

    return sweep


_edge_sweep_128 = _make_edge_sweep(D)
_edge_sweep_16 = _make_edge_sweep(L)


def _pack(al, ar):
    H, Fo = al.shape
    eye = jnp.eye(H, 8, dtype=jnp.float32)
    a_el = (al[:, :, None] * eye[:, None, :]).reshape(H * Fo, 8)
    a_er = (ar[:, :, None] * eye[:, None, :]).reshape(H * Fo, 8)
    return (jnp.concatenate([a_el, a_er], 1).astype(jnp.float32),
            jnp.concatenate([a_er, a_el], 1).astype(jnp.float32))


def _mtile(mx, H):
    m = mx[0]
    s = m[:8] + m[8:]
    s = jnp.where(s > 0, s, 0.2 * s)
    head = jnp.where(jnp.arange(8) < H, s, 1e30)
    return jnp.concatenate([head, jnp.full((8,), 1e30, jnp.float32)])


def kernel(feats, edge_index, W1, al1, ar1, b1, W2, al2, ar2, b2):
    pe1, pr1 = _pack(al1, ar1)
    pe2, pr2 = _pack(al2, ar2)
    q = (jnp.eye(L, 8, dtype=jnp.float32)[:, :, None]
         * jnp.ones((1, 1, L), jnp.float32)).reshape(L, D)
    q2 = jnp.zeros((L, L), jnp.float32).at[0, :].set(1.0)

    src = edge_index[0]
    dst = edge_index[1]
    h1, elr1, rle1, mx1 = _tc_project(feats, W1, pe1, pr1)
    m1 = _mtile(mx1, 8)
    out1p, den1p = _edge_sweep_128(src, dst, h1, elr1, rle1, m1)

    h2, elr2, rle2, mx2 = _tc_combine_project(
        out1p, den1p, W2, q, b1.reshape(1, D), pe2, pr2)
    m2 = _mtile(mx2, 1)
    out2p, den2p = _edge_sweep_16(src, dst, h2, elr2, rle2, m2)

    return _tc_finish(out2p, den2p, q2, b2.reshape(1, L))


# speedup vs baseline: 2.4286x; 1.2971x over previous
import functools

import jax
import jax.numpy as jnp
from jax import lax
from jax.experimental import pallas as pl
from jax.experimental.pallas import tpu as pltpu
from jax.experimental.pallas import tpu_sc as plsc

N = 10000
E = 320000
D = 128
NC, NS, L = 2, 16, 16
NW = NC * NS
EPW = E // NW
K = 40
NB = EPW // K
ROWS_T = 624
TAIL = N - NS * ROWS_T
BN = 1000


def _tc1_body(x_ref, w_ref, pe_ref, pr_ref, h_ref, elr_ref, rle_ref, mx_ref):
    i = pl.program_id(0)
    h = jnp.dot(x_ref[...], w_ref[...], preferred_element_type=jnp.float32)
    h_ref[...] = h
    t = jnp.dot(h, pe_ref[...], preferred_element_type=jnp.float32)
    elr_ref[...] = t
    rle_ref[...] = jnp.dot(h, pr_ref[...], preferred_element_type=jnp.float32)

    @pl.when(i == 0)
    def _():
        mx_ref[...] = jnp.full((1, L), -1e30, jnp.float32)

    mx_ref[...] = jnp.maximum(mx_ref[...], jnp.max(t, axis=0, keepdims=True))


def _tc_project(x, W, Pe, Pr):
    n, d = x.shape
    r = W.shape[1]
    return pl.pallas_call(
        _tc1_body,
        grid=(n // BN,),
        in_specs=[pl.BlockSpec((BN, d), lambda i: (i, 0)),
                  pl.BlockSpec((d, r), lambda i: (0, 0)),
                  pl.BlockSpec((r, L), lambda i: (0, 0)),
                  pl.BlockSpec((r, L), lambda i: (0, 0))],
        out_specs=[pl.BlockSpec((BN, r), lambda i: (i, 0)),
                   pl.BlockSpec((BN, L), lambda i: (i, 0)),
                   pl.BlockSpec((BN, L), lambda i: (i, 0)),
                   pl.BlockSpec((1, L), lambda i: (0, 0))],
        out_shape=[jax.ShapeDtypeStruct((n, r), jnp.float32),
                   jax.ShapeDtypeStruct((n, L), jnp.float32),
                   jax.ShapeDtypeStruct((n, L), jnp.float32),
                   jax.ShapeDtypeStruct((1, L), jnp.float32)],
    )(x, W, Pe, Pr)


def _tc2_body(p_ref, d_ref, w_ref, q_ref, b_ref, pe_ref, pr_ref,
              h2_ref, elr_ref, rle_ref, mx_ref):
    i = pl.program_id(0)
    num = p_ref[0] + p_ref[1]
    den = d_ref[0] + d_ref[1]
    den128 = jnp.dot(den, q_ref[...], preferred_element_type=jnp.float32)
    x2 = num / (den128 + 1e-9) + b_ref[...]
    x2 = jnp.maximum(x2, 0.0)
    h2 = jnp.dot(x2, w_ref[...], preferred_element_type=jnp.float32)
    h2_ref[...] = h2
    t = jnp.dot(h2, pe_ref[...], preferred_element_type=jnp.float32)
    elr_ref[...] = t
    rle_ref[...] = jnp.dot(h2, pr_ref[...], preferred_element_type=jnp.float32)

    @pl.when(i == 0)
    def _():
        mx_ref[...] = jnp.full((1, L), -1e30, jnp.float32)

    mx_ref[...] = jnp.maximum(mx_ref[...], jnp.max(t, axis=0, keepdims=True))


def _tc_combine_project(outp, denp, W2, Q, b1row, Pe, Pr):
    return pl.pallas_call(
        _tc2_body,
        grid=(N // BN,),
        in_specs=[pl.BlockSpec((NC, BN, D), lambda i: (0, i, 0)),
                  pl.BlockSpec((NC, BN, L), lambda i: (0, i, 0)),
                  pl.BlockSpec((D, L), lambda i: (0, 0)),
                  pl.BlockSpec((L, D), lambda i: (0, 0)),
                  pl.BlockSpec((1, D), lambda i: (0, 0)),
                  pl.BlockSpec((L, L), lambda i: (0, 0)),
                  pl.BlockSpec((L, L), lambda i: (0, 0))],
        out_specs=[pl.BlockSpec((BN, L), lambda i: (i, 0)),
                   pl.BlockSpec((BN, L), lambda i: (i, 0)),
                   pl.BlockSpec((BN, L), lambda i: (i, 0)),
                   pl.BlockSpec((1, L), lambda i: (0, 0))],
        out_shape=[jax.ShapeDtypeStruct((N, L), jnp.float32),
                   jax.ShapeDtypeStruct((N, L), jnp.float32),
                   jax.ShapeDtypeStruct((N, L), jnp.float32),
                   jax.ShapeDtypeStruct((1, L), jnp.float32)],
    )(outp, denp, W2, Q, b1row, Pe, Pr)


def _tc3_body(p_ref, d_ref, q2_ref, b_ref, o_ref):
    num = p_ref[0] + p_ref[1]
    den = d_ref[0] + d_ref[1]
    den16 = jnp.dot(den, q2_ref[...], preferred_element_type=jnp.float32)
    o_ref[...] = num / (den16 + 1e-9) + b_ref[...]


def _tc_finish(outp, denp, Q2, b2row):
    return pl.pallas_call(
        _tc3_body,
        grid=(N // BN,),
        in_specs=[pl.BlockSpec((NC, BN, L), lambda i: (0, i, 0)),
                  pl.BlockSpec((NC, BN, L), lambda i: (0, i, 0)),
                  pl.BlockSpec((L, L), lambda i: (0, 0)),
                  pl.BlockSpec((1, L), lambda i: (0, 0))],
        out_specs=pl.BlockSpec((BN, L), lambda i: (i, 0)),
        out_shape=jax.ShapeDtypeStruct((N, L), jnp.float32),
    )(outp, denp, Q2, b2row)


_BCAST_DNUMS = lax.GatherDimensionNumbers(
    offset_dims=(), collapsed_slice_dims=(0,), start_index_map=(0,))


def _lane_bcast(v, j):
    idx = jnp.full((L, 1), j, jnp.int32)
    return lax.gather(v, idx, _BCAST_DNUMS, (1,),
                      mode=lax.GatherScatterMode.PROMISE_IN_BOUNDS)


def _make_edge_sweep(R):
    RC = R // L
    mesh = plsc.VectorSubcoreMesh(core_axis_name="c", subcore_axis_name="s")

    slot_types = (
        pltpu.VMEM((K,), jnp.int32),
        pltpu.VMEM((K, L), jnp.float32),
        pltpu.VMEM((K, L), jnp.float32),
        pltpu.VMEM((K, L), jnp.float32),
        pltpu.VMEM((K, R), jnp.float32),
        pltpu.VMEM((K, R), jnp.float32),
        pltpu.SemaphoreType.DMA,
        pltpu.SemaphoreType.DMA,
    )

    @functools.partial(
        pl.kernel,
        out_type=(jax.ShapeDtypeStruct((NC, N, R), jnp.float32),
                  jax.ShapeDtypeStruct((NC, N, L), jnp.float32)),
        mesh=mesh,
        compiler_params=pltpu.CompilerParams(use_tc_tiling_on_sc=False),
        scratch_types=(
            pltpu.VMEM_SHARED((N, R), jnp.float32),
            pltpu.VMEM_SHARED((N, L), jnp.float32),
            pltpu.VMEM((L,), jnp.float32),
            pltpu.VMEM((NB, K), jnp.int32),
        ) + slot_types + slot_types,
    )
    def sweep(src_hbm, dst2_hbm, h_hbm, elr_hbm, rle_hbm, m_hbm,
              out_hbm, den_hbm,
              out_sp, den_sp, m_v, dst_all, *slot_refs):
        ns = len(slot_types)
        slots = (slot_refs[:ns], slot_refs[ns:])
        cid = lax.axis_index("c")
        tid = lax.axis_index("s")
        wid = cid * NS + tid
        rbase = tid * ROWS_T

        ebase = wid * EPW
        pltpu.sync_copy(dst2_hbm.at[pl.ds(wid * NB, NB)], dst_all)

        zmsg = slots[0][5]
        zw = slots[0][3]

        def zo(i, c):
            for j in range(RC):
                zmsg[i, pl.ds(j * L, L)] = jnp.zeros((L,), jnp.float32)
            zw[i, :] = jnp.zeros((L,), jnp.float32)
            return c

        lax.fori_loop(0, K, zo, 0, unroll=4)
        for z in range(ROWS_T // K):
            pltpu.sync_copy(zmsg, out_sp.at[pl.ds(rbase + z * K, K)])
            pltpu.sync_copy(zw, den_sp.at[pl.ds(rbase + z * K, K)])
        rem = ROWS_T - (ROWS_T // K) * K
        pltpu.sync_copy(zmsg.at[pl.ds(0, rem)],
                        out_sp.at[pl.ds(rbase + ROWS_T - rem, rem)])
        pltpu.sync_copy(zw.at[pl.ds(0, rem)],
                        den_sp.at[pl.ds(rbase + ROWS_T - rem, rem)])

        @pl.when(tid == NS - 1)
        def _():
            tb = NS * ROWS_T
            pltpu.sync_copy(zmsg.at[pl.ds(0, TAIL)],
                            out_sp.at[pl.ds(tb, TAIL)])
            pltpu.sync_copy(zw.at[pl.ds(0, TAIL)],
                            den_sp.at[pl.ds(tb, TAIL)])

        pltpu.sync_copy(m_hbm, m_v)
        plsc.subcore_barrier()

        mvec = m_v[...]
        lane = lax.iota(jnp.int32, L)
        headmask = lane < 8

        def issue_gathers(s, b):
            src_v, elrs_v, rled_v, _, rows_v, _, semg, _ = slots[s]
            gb = pl.multiple_of(ebase + b * K, 8)
            pltpu.sync_copy(src_hbm.at[pl.ds(gb, K)], src_v)
            pltpu.async_copy(elr_hbm.at[src_v], elrs_v, semg)
            pltpu.async_copy(rle_hbm.at[dst_all.at[b]], rled_v, semg)
            pltpu.async_copy(h_hbm.at[src_v], rows_v, semg)

        def wait_gathers(s, b):
            src_v, elrs_v, rled_v, _, rows_v, _, semg, _ = slots[s]
            pltpu.make_async_copy(elr_hbm.at[src_v], elrs_v, semg).wait()
            pltpu.make_async_copy(rle_hbm.at[dst_all.at[b]],
                                  rled_v, semg).wait()
            pltpu.make_async_copy(h_hbm.at[src_v], rows_v, semg).wait()

        def wait_scatters(s, b):
            _, _, _, w_v, _, msg_v, _, sems = slots[s]
            pltpu.make_async_copy(w_v, den_sp.at[dst_all.at[b]], sems).wait()
            pltpu.make_async_copy(msg_v, out_sp.at[dst_all.at[b]], sems).wait()

        def compute_and_scatter(s, b):
            _, elrs_v, rled_v, w_v, rows_v, msg_v, _, sems = slots[s]

            @plsc.parallel_loop(0, K, unroll=(4 if RC == 1 else 2))
            def _(k):
                e = elrs_v[k, :] + rled_v[k, :]
                e = jnp.where(e > 0, e, 0.2 * e)
                wrow = jnp.where(headmask, jnp.exp(e - mvec), 0.0)
                w_v[k, :] = wrow
                for j in range(RC):
                    msg_v[k, pl.ds(j * L, L)] = (
                        rows_v[k, pl.ds(j * L, L)] * _lane_bcast(wrow, j))

            pltpu.async_copy(w_v, den_sp.at[dst_all.at[b]], sems, add=True)
            pltpu.async_copy(msg_v, out_sp.at[dst_all.at[b]], sems, add=True)

        NPAIRS = NB // 2

        issue_gathers(0, 0)
        issue_gathers(1, 1)

        def pair(p, c):
            wait_gathers(0, 2 * p)

            @pl.when(p > 0)
            def _():
                wait_scatters(0, 2 * p - 2)

            compute_and_scatter(0, 2 * p)

            @pl.when(2 * p + 2 < NB)
            def _():
                issue_gathers(0, 2 * p + 2)

            wait_gathers(1, 2 * p + 1)

            @pl.when(p > 0)
            def _():
                wait_scatters(1, 2 * p - 1)

            compute_and_scatter(1, 2 * p + 1)

            @pl.when(2 * p + 3 < NB)
            def _():
                issue_gathers(1, 2 * p + 3)

            return c

        lax.fori_loop(0, NPAIRS, pair, 0)

        wait_scatters(0, NB - 2)
        wait_scatters(1, NB - 1)

        plsc.subcore_barrier()
        pltpu.sync_copy(out_sp.at[pl.ds(rbase, ROWS_T)],
                        out_hbm.at[cid, pl.ds(rbase, ROWS_T)])
        pltpu.sync_copy(den_sp.at[pl.ds(rbase, ROWS_T)],
                        den_hbm.at[cid, pl.ds(rbase, ROWS_T)])

        @pl.when(tid == NS - 1)
        def _():
            tb = NS * ROWS_T
            pltpu.sync_copy(out_sp.at[pl.ds(tb, TAIL)],
                            out_hbm.at[cid, pl.ds(tb, TAIL)])
            pltpu.sync_copy(den_sp.at[pl.ds(tb, TAIL)],
                            den_hbm.at[cid, pl.ds(tb, TAIL)])

    return sweep


_edge_sweep_128 = _make_edge_sweep(D)
_edge_sweep_16 = _make_edge_sweep(L)


def _pack(al, ar):
    H, Fo = al.shape
    eye = jnp.eye(H, 8, dtype=jnp.float32)
    a_el = (al[:, :, None] * eye[:, None, :]).reshape(H * Fo, 8)
    a_er = (ar[:, :, None] * eye[:, None, :]).reshape(H * Fo, 8)
    return (jnp.concatenate([a_el, a_er], 1).astype(jnp.float32),
            jnp.concatenate([a_er, a_el], 1).astype(jnp.float32))


def _mtile(mx, H):
    m = mx[0]
    s = m[:8] + m[8:]
    s = jnp.where(s > 0, s, 0.2 * s)
    head = jnp.where(jnp.arange(8) < H, s, 1e30)
    return jnp.concatenate([head, jnp.full((8,), 1e30, jnp.float32)])


def kernel(feats, edge_index, W1, al1, ar1, b1, W2, al2, ar2, b2):
    pe1, pr1 = _pack(al1, ar1)
    pe2, pr2 = _pack(al2, ar2)
    q = (jnp.eye(L, 8, dtype=jnp.float32)[:, :, None]
         * jnp.ones((1, 1, L), jnp.float32)).reshape(L, D)
    q2 = jnp.zeros((L, L), jnp.float32).at[0, :].set(1.0)

    src = edge_index[0]
    dst2 = edge_index[1].reshape(NW * NB, K)
    h1, elr1, rle1, mx1 = _tc_project(feats, W1, pe1, pr1)
    m1 = _mtile(mx1, 8)
    out1p, den1p = _edge_sweep_128(src, dst2, h1, elr1, rle1, m1)

    h2, elr2, rle2, mx2 = _tc_combine_project(
        out1p, den1p, W2, q, b1.reshape(1, D), pe2, pr2)
    m2 = _mtile(mx2, 1)
    out2p, den2p = _edge_sweep_16(src, dst2, h2, elr2, rle2, m2)

    return _tc_finish(out2p, den2p, q2, b2.reshape(1, L))


# speedup vs baseline: 2.6673x; 1.0983x over previous
import functools

import jax
import jax.numpy as jnp
from jax import lax
from jax.experimental import pallas as pl
from jax.experimental.pallas import tpu as pltpu
from jax.experimental.pallas import tpu_sc as plsc

N = 10000
E = 320000
D = 128
NC, NS, L = 2, 16, 16
NW = NC * NS
EPW = E // NW
K = 40
NB = EPW // K
ROWS_T = 624
TAIL = N - NS * ROWS_T
BN = 1000


def _tc1_body(x_ref, w_ref, pe_ref, pr_ref, h_ref, elr_ref, rle_ref, mx_ref):
    i = pl.program_id(0)
    h = jnp.dot(x_ref[...], w_ref[...], preferred_element_type=jnp.float32)
    h_ref[...] = h
    t = jnp.dot(h, pe_ref[...], preferred_element_type=jnp.float32)
    elr_ref[...] = t
    rle_ref[...] = jnp.dot(h, pr_ref[...], preferred_element_type=jnp.float32)

    @pl.when(i == 0)
    def _():
        mx_ref[...] = jnp.full((1, L), -1e30, jnp.float32)

    mx_ref[...] = jnp.maximum(mx_ref[...], jnp.max(t, axis=0, keepdims=True))


def _tc_project(x, W, Pe, Pr):
    n, d = x.shape
    r = W.shape[1]
    return pl.pallas_call(
        _tc1_body,
        grid=(n // BN,),
        in_specs=[pl.BlockSpec((BN, d), lambda i: (i, 0)),
                  pl.BlockSpec((d, r), lambda i: (0, 0)),
                  pl.BlockSpec((r, L), lambda i: (0, 0)),
                  pl.BlockSpec((r, L), lambda i: (0, 0))],
        out_specs=[pl.BlockSpec((BN, r), lambda i: (i, 0)),
                   pl.BlockSpec((BN, L), lambda i: (i, 0)),
                   pl.BlockSpec((BN, L), lambda i: (i, 0)),
                   pl.BlockSpec((1, L), lambda i: (0, 0))],
        out_shape=[jax.ShapeDtypeStruct((n, r), jnp.float32),
                   jax.ShapeDtypeStruct((n, L), jnp.float32),
                   jax.ShapeDtypeStruct((n, L), jnp.float32),
                   jax.ShapeDtypeStruct((1, L), jnp.float32)],
    )(x, W, Pe, Pr)


def _tc2_body(p_ref, d_ref, w_ref, q_ref, b_ref, pe_ref, pr_ref,
              h2_ref, elr_ref, rle_ref, mx_ref):
    i = pl.program_id(0)
    num = p_ref[0] + p_ref[1]
    den = d_ref[0] + d_ref[1]
    den128 = jnp.dot(den, q_ref[...], preferred_element_type=jnp.float32)
    x2 = num / (den128 + 1e-9) + b_ref[...]
    x2 = jnp.maximum(x2, 0.0)
    h2 = jnp.dot(x2, w_ref[...], preferred_element_type=jnp.float32)
    h2_ref[...] = h2
    t = jnp.dot(h2, pe_ref[...], preferred_element_type=jnp.float32)
    elr_ref[...] = t
    rle_ref[...] = jnp.dot(h2, pr_ref[...], preferred_element_type=jnp.float32)

    @pl.when(i == 0)
    def _():
        mx_ref[...] = jnp.full((1, L), -1e30, jnp.float32)

    mx_ref[...] = jnp.maximum(mx_ref[...], jnp.max(t, axis=0, keepdims=True))


def _tc_combine_project(outp, denp, W2, Q, b1row, Pe, Pr):
    return pl.pallas_call(
        _tc2_body,
        grid=(N // BN,),
        in_specs=[pl.BlockSpec((NC, BN, D), lambda i: (0, i, 0)),
                  pl.BlockSpec((NC, BN, L), lambda i: (0, i, 0)),
                  pl.BlockSpec((D, L), lambda i: (0, 0)),
                  pl.BlockSpec((L, D), lambda i: (0, 0)),
                  pl.BlockSpec((1, D), lambda i: (0, 0)),
                  pl.BlockSpec((L, L), lambda i: (0, 0)),
                  pl.BlockSpec((L, L), lambda i: (0, 0))],
        out_specs=[pl.BlockSpec((BN, L), lambda i: (i, 0)),
                   pl.BlockSpec((BN, L), lambda i: (i, 0)),
                   pl.BlockSpec((BN, L), lambda i: (i, 0)),
                   pl.BlockSpec((1, L), lambda i: (0, 0))],
        out_shape=[jax.ShapeDtypeStruct((N, L), jnp.float32),
                   jax.ShapeDtypeStruct((N, L), jnp.float32),
                   jax.ShapeDtypeStruct((N, L), jnp.float32),
                   jax.ShapeDtypeStruct((1, L), jnp.float32)],
    )(outp, denp, W2, Q, b1row, Pe, Pr)


def _tc3_body(p_ref, d_ref, q2_ref, b_ref, o_ref):
    num = p_ref[0] + p_ref[1]
    den = d_ref[0] + d_ref[1]
    den16 = jnp.dot(den, q2_ref[...], preferred_element_type=jnp.float32)
    o_ref[...] = num / (den16 + 1e-9) + b_ref[...]


def _tc_finish(outp, denp, Q2, b2row):
    return pl.pallas_call(
        _tc3_body,
        grid=(N // BN,),
        in_specs=[pl.BlockSpec((NC, BN, L), lambda i: (0, i, 0)),
                  pl.BlockSpec((NC, BN, L), lambda i: (0, i, 0)),
                  pl.BlockSpec((L, L), lambda i: (0, 0)),
                  pl.BlockSpec((1, L), lambda i: (0, 0))],
        out_specs=pl.BlockSpec((BN, L), lambda i: (i, 0)),
        out_shape=jax.ShapeDtypeStruct((N, L), jnp.float32),
    )(outp, denp, Q2, b2row)


_BCAST_DNUMS = lax.GatherDimensionNumbers(
    offset_dims=(), collapsed_slice_dims=(0,), start_index_map=(0,))


def _lane_bcast(v, j):
    idx = jnp.full((L, 1), j, jnp.int32)
    return lax.gather(v, idx, _BCAST_DNUMS, (1,),
                      mode=lax.GatherScatterMode.PROMISE_IN_BOUNDS)


def _make_edge_sweep(R):
    RC = R // L
    mesh = plsc.VectorSubcoreMesh(core_axis_name="c", subcore_axis_name="s")

    slot_types = (
        pltpu.VMEM((K,), jnp.int32),
        pltpu.VMEM((K, L), jnp.float32),
        pltpu.VMEM((K, L), jnp.float32),
        pltpu.VMEM((K, L), jnp.float32),
        pltpu.VMEM((K, R), jnp.float32),
        pltpu.VMEM((K, R), jnp.float32),
        pltpu.SemaphoreType.DMA,
        pltpu.SemaphoreType.DMA,
        pltpu.SemaphoreType.DMA,
    )

    @functools.partial(
        pl.kernel,
        out_type=(jax.ShapeDtypeStruct((NC, N, R), jnp.float32),
                  jax.ShapeDtypeStruct((NC, N, L), jnp.float32)),
        mesh=mesh,
        compiler_params=pltpu.CompilerParams(use_tc_tiling_on_sc=False),
        scratch_types=(
            pltpu.VMEM_SHARED((N, R), jnp.float32),
            pltpu.VMEM_SHARED((N, L), jnp.float32),
            pltpu.VMEM((L,), jnp.float32),
            pltpu.VMEM((NB, K), jnp.int32),
        ) + slot_types + slot_types,
    )
    def sweep(src_hbm, dst2_hbm, h_hbm, elr_hbm, rle_hbm, m_hbm,
              out_hbm, den_hbm,
              out_sp, den_sp, m_v, dst_all, *slot_refs):
        ns = len(slot_types)
        slots = (slot_refs[:ns], slot_refs[ns:])
        cid = lax.axis_index("c")
        tid = lax.axis_index("s")
        wid = cid * NS + tid
        rbase = tid * ROWS_T

        ebase = wid * EPW
        pltpu.sync_copy(dst2_hbm.at[pl.ds(wid * NB, NB)], dst_all)

        zmsg = slots[0][5]
        zw = slots[0][3]

        def zo(i, c):
            for j in range(RC):
                zmsg[i, pl.ds(j * L, L)] = jnp.zeros((L,), jnp.float32)
            zw[i, :] = jnp.zeros((L,), jnp.float32)
            return c

        lax.fori_loop(0, K, zo, 0, unroll=4)
        for z in range(ROWS_T // K):
            pltpu.sync_copy(zmsg, out_sp.at[pl.ds(rbase + z * K, K)])
            pltpu.sync_copy(zw, den_sp.at[pl.ds(rbase + z * K, K)])
        rem = ROWS_T - (ROWS_T // K) * K
        pltpu.sync_copy(zmsg.at[pl.ds(0, rem)],
                        out_sp.at[pl.ds(rbase + ROWS_T - rem, rem)])
        pltpu.sync_copy(zw.at[pl.ds(0, rem)],
                        den_sp.at[pl.ds(rbase + ROWS_T - rem, rem)])

        @pl.when(tid == NS - 1)
        def _():
            tb = NS * ROWS_T
            pltpu.sync_copy(zmsg.at[pl.ds(0, TAIL)],
                            out_sp.at[pl.ds(tb, TAIL)])
            pltpu.sync_copy(zw.at[pl.ds(0, TAIL)],
                            den_sp.at[pl.ds(tb, TAIL)])

        pltpu.sync_copy(m_hbm, m_v)
        plsc.subcore_barrier()

        mvec = m_v[...]
        lane = lax.iota(jnp.int32, L)
        headmask = lane < 8

        def prefetch_src(s, b):
            src_v, _, _, _, _, _, semi, _, _ = slots[s]
            gb = pl.multiple_of(ebase + b * K, 8)
            pltpu.async_copy(src_hbm.at[pl.ds(gb, K)], src_v, semi)

        def issue_gathers(s, b):
            src_v, elrs_v, rled_v, _, rows_v, _, semi, semg, _ = slots[s]
            gb = pl.multiple_of(ebase + b * K, 8)
            pltpu.make_async_copy(src_hbm.at[pl.ds(gb, K)], src_v, semi).wait()
            pltpu.async_copy(elr_hbm.at[src_v], elrs_v, semg)
            pltpu.async_copy(rle_hbm.at[dst_all.at[b]], rled_v, semg)
            pltpu.async_copy(h_hbm.at[src_v], rows_v, semg)

        def wait_gathers(s, b):
            src_v, elrs_v, rled_v, _, rows_v, _, _, semg, _ = slots[s]
            pltpu.make_async_copy(elr_hbm.at[src_v], elrs_v, semg).wait()
            pltpu.make_async_copy(rle_hbm.at[dst_all.at[b]],
                                  rled_v, semg).wait()
            pltpu.make_async_copy(h_hbm.at[src_v], rows_v, semg).wait()

        def wait_scatters(s, b):
            _, _, _, w_v, _, msg_v, _, _, sems = slots[s]
            pltpu.make_async_copy(w_v, den_sp.at[dst_all.at[b]], sems).wait()
            pltpu.make_async_copy(msg_v, out_sp.at[dst_all.at[b]], sems).wait()

        def compute_and_scatter(s, b):
            _, elrs_v, rled_v, w_v, rows_v, msg_v, _, _, sems = slots[s]

            @plsc.parallel_loop(0, K, unroll=(4 if RC == 1 else 2))
            def _(k):
                e = elrs_v[k, :] + rled_v[k, :]
                e = jnp.where(e > 0, e, 0.2 * e)
                wrow = jnp.where(headmask, jnp.exp(e - mvec), 0.0)
                w_v[k, :] = wrow
                for j in range(RC):
                    msg_v[k, pl.ds(j * L, L)] = (
                        rows_v[k, pl.ds(j * L, L)] * _lane_bcast(wrow, j))

            pltpu.async_copy(w_v, den_sp.at[dst_all.at[b]], sems, add=True)
            pltpu.async_copy(msg_v, out_sp.at[dst_all.at[b]], sems, add=True)

        NPAIRS = NB // 2

        prefetch_src(0, 0)
        prefetch_src(1, 1)
        issue_gathers(0, 0)
        issue_gathers(1, 1)

        def pair(p, c):
            wait_gathers(0, 2 * p)

            @pl.when(2 * p + 2 < NB)
            def _():
                prefetch_src(0, 2 * p + 2)

            @pl.when(p > 0)
            def _():
                wait_scatters(0, 2 * p - 2)

            compute_and_scatter(0, 2 * p)

            @pl.when(2 * p + 2 < NB)
            def _():
                issue_gathers(0, 2 * p + 2)

            wait_gathers(1, 2 * p + 1)

            @pl.when(2 * p + 3 < NB)
            def _():
                prefetch_src(1, 2 * p + 3)

            @pl.when(p > 0)
            def _():
                wait_scatters(1, 2 * p - 1)

            compute_and_scatter(1, 2 * p + 1)

            @pl.when(2 * p + 3 < NB)
            def _():
                issue_gathers(1, 2 * p + 3)

            return c

        lax.fori_loop(0, NPAIRS, pair, 0)

        wait_scatters(0, NB - 2)
        wait_scatters(1, NB - 1)

        plsc.subcore_barrier()
        pltpu.sync_copy(out_sp.at[pl.ds(rbase, ROWS_T)],
                        out_hbm.at[cid, pl.ds(rbase, ROWS_T)])
        pltpu.sync_copy(den_sp.at[pl.ds(rbase, ROWS_T)],
                        den_hbm.at[cid, pl.ds(rbase, ROWS_T)])

        @pl.when(tid == NS - 1)
        def _():
            tb = NS * ROWS_T
            pltpu.sync_copy(out_sp.at[pl.ds(tb, TAIL)],
                            out_hbm.at[cid, pl.ds(tb, TAIL)])
            pltpu.sync_copy(den_sp.at[pl.ds(tb, TAIL)],
                            den_hbm.at[cid, pl.ds(tb, TAIL)])

    return sweep


_edge_sweep_128 = _make_edge_sweep(D)
_edge_sweep_16 = _make_edge_sweep(L)


def _pack(al, ar):
    H, Fo = al.shape
    eye = jnp.eye(H, 8, dtype=jnp.float32)
    a_el = (al[:, :, None] * eye[:, None, :]).reshape(H * Fo, 8)
    a_er = (ar[:, :, None] * eye[:, None, :]).reshape(H * Fo, 8)
    return (jnp.concatenate([a_el, a_er], 1).astype(jnp.float32),
            jnp.concatenate([a_er, a_el], 1).astype(jnp.float32))


def _mtile(mx, H):
    m = mx[0]
    s = m[:8] + m[8:]
    s = jnp.where(s > 0, s, 0.2 * s)
    head = jnp.where(jnp.arange(8) < H, s, 1e30)
    return jnp.concatenate([head, jnp.full((8,), 1e30, jnp.float32)])


def kernel(feats, edge_index, W1, al1, ar1, b1, W2, al2, ar2, b2):
    pe1, pr1 = _pack(al1, ar1)
    pe2, pr2 = _pack(al2, ar2)
    q = (jnp.eye(L, 8, dtype=jnp.float32)[:, :, None]
         * jnp.ones((1, 1, L), jnp.float32)).reshape(L, D)
    q2 = jnp.zeros((L, L), jnp.float32).at[0, :].set(1.0)

    src = edge_index[0]
    dst2 = edge_index[1].reshape(NW * NB, K)
    h1, elr1, rle1, mx1 = _tc_project(feats, W1, pe1, pr1)
    m1 = _mtile(mx1, 8)
    out1p, den1p = _edge_sweep_128(src, dst2, h1, elr1, rle1, m1)

    h2, elr2, rle2, mx2 = _tc_combine_project(
        out1p, den1p, W2, q, b1.reshape(1, D), pe2, pr2)
    m2 = _mtile(mx2, 1)
    out2p, den2p = _edge_sweep_16(src, dst2, h2, elr2, rle2, m2)

    return _tc_finish(out2p, den2p, q2, b2.reshape(1, L))


# speedup vs baseline: 3.0558x; 1.1457x over previous
import functools

import jax
import jax.numpy as jnp
from jax import lax
from jax.experimental import pallas as pl
from jax.experimental.pallas import tpu as pltpu
from jax.experimental.pallas import tpu_sc as plsc

N = 10000
E = 320000
D = 128
NC, NS, L = 2, 16, 16
NW = NC * NS
EPW = E // NW
K1 = 40
K2 = 80
ROWS_T = 624
TAIL = N - NS * ROWS_T
BN = 1000


def _tc1_body(x_ref, w_ref, pe_ref, pr_ref, h_ref, elr_ref, rle_ref, mx_ref):
    i = pl.program_id(0)
    h = jnp.dot(x_ref[...], w_ref[...], preferred_element_type=jnp.float32)
    h_ref[...] = h
    t = jnp.dot(h, pe_ref[...], preferred_element_type=jnp.float32)
    elr_ref[...] = t
    rle_ref[...] = jnp.dot(h, pr_ref[...], preferred_element_type=jnp.float32)

    @pl.when(i == 0)
    def _():
        mx_ref[...] = jnp.full((1, L), -1e30, jnp.float32)

    mx_ref[...] = jnp.maximum(mx_ref[...], jnp.max(t, axis=0, keepdims=True))


def _tc_project(x, W, Pe, Pr):
    n, d = x.shape
    r = W.shape[1]
    return pl.pallas_call(
        _tc1_body,
        grid=(n // BN,),
        in_specs=[pl.BlockSpec((BN, d), lambda i: (i, 0)),
                  pl.BlockSpec((d, r), lambda i: (0, 0)),
                  pl.BlockSpec((r, L), lambda i: (0, 0)),
                  pl.BlockSpec((r, L), lambda i: (0, 0))],
        out_specs=[pl.BlockSpec((BN, r), lambda i: (i, 0)),
                   pl.BlockSpec((BN, L), lambda i: (i, 0)),
                   pl.BlockSpec((BN, L), lambda i: (i, 0)),
                   pl.BlockSpec((1, L), lambda i: (0, 0))],
        out_shape=[jax.ShapeDtypeStruct((n, r), jnp.float32),
                   jax.ShapeDtypeStruct((n, L), jnp.float32),
                   jax.ShapeDtypeStruct((n, L), jnp.float32),
                   jax.ShapeDtypeStruct((1, L), jnp.float32)],
    )(x, W, Pe, Pr)


def _tc2_body(p_ref, d_ref, w_ref, q_ref, b_ref, pe_ref, pr_ref,
              h2_ref, elr_ref, rle_ref, mx_ref):
    i = pl.program_id(0)
    num = p_ref[0] + p_ref[1]
    den = d_ref[0] + d_ref[1]
    den128 = jnp.dot(den, q_ref[...], preferred_element_type=jnp.float32)
    x2 = num / (den128 + 1e-9) + b_ref[...]
    x2 = jnp.maximum(x2, 0.0)
    h2 = jnp.dot(x2, w_ref[...], preferred_element_type=jnp.float32)
    h2_ref[...] = h2
    t = jnp.dot(h2, pe_ref[...], preferred_element_type=jnp.float32)
    elr_ref[...] = t
    rle_ref[...] = jnp.dot(h2, pr_ref[...], preferred_element_type=jnp.float32)

    @pl.when(i == 0)
    def _():
        mx_ref[...] = jnp.full((1, L), -1e30, jnp.float32)

    mx_ref[...] = jnp.maximum(mx_ref[...], jnp.max(t, axis=0, keepdims=True))


def _tc_combine_project(outp, denp, W2, Q, b1row, Pe, Pr):
    return pl.pallas_call(
        _tc2_body,
        grid=(N // BN,),
        in_specs=[pl.BlockSpec((NC, BN, D), lambda i: (0, i, 0)),
                  pl.BlockSpec((NC, BN, L), lambda i: (0, i, 0)),
                  pl.BlockSpec((D, L), lambda i: (0, 0)),
                  pl.BlockSpec((L, D), lambda i: (0, 0)),
                  pl.BlockSpec((1, D), lambda i: (0, 0)),
                  pl.BlockSpec((L, L), lambda i: (0, 0)),
                  pl.BlockSpec((L, L), lambda i: (0, 0))],
        out_specs=[pl.BlockSpec((BN, L), lambda i: (i, 0)),
                   pl.BlockSpec((BN, L), lambda i: (i, 0)),
                   pl.BlockSpec((BN, L), lambda i: (i, 0)),
                   pl.BlockSpec((1, L), lambda i: (0, 0))],
        out_shape=[jax.ShapeDtypeStruct((N, L), jnp.float32),
                   jax.ShapeDtypeStruct((N, L), jnp.float32),
                   jax.ShapeDtypeStruct((N, L), jnp.float32),
                   jax.ShapeDtypeStruct((1, L), jnp.float32)],
    )(outp, denp, W2, Q, b1row, Pe, Pr)


def _tc3_body(p_ref, d_ref, q2_ref, b_ref, o_ref):
    num = p_ref[0] + p_ref[1]
    den = d_ref[0] + d_ref[1]
    den16 = jnp.dot(den, q2_ref[...], preferred_element_type=jnp.float32)
    o_ref[...] = num / (den16 + 1e-9) + b_ref[...]


def _tc_finish(outp, denp, Q2, b2row):
    return pl.pallas_call(
        _tc3_body,
        grid=(N // BN,),
        in_specs=[pl.BlockSpec((NC, BN, L), lambda i: (0, i, 0)),
                  pl.BlockSpec((NC, BN, L), lambda i: (0, i, 0)),
                  pl.BlockSpec((L, L), lambda i: (0, 0)),
                  pl.BlockSpec((1, L), lambda i: (0, 0))],
        out_specs=pl.BlockSpec((BN, L), lambda i: (i, 0)),
        out_shape=jax.ShapeDtypeStruct((N, L), jnp.float32),
    )(outp, denp, Q2, b2row)


_BCAST_DNUMS = lax.GatherDimensionNumbers(
    offset_dims=(), collapsed_slice_dims=(0,), start_index_map=(0,))


def _lane_bcast(v, j):
    idx = jnp.full((L, 1), j, jnp.int32)
    return lax.gather(v, idx, _BCAST_DNUMS, (1,),
                      mode=lax.GatherScatterMode.PROMISE_IN_BOUNDS)


def _make_edge_sweep(R, K):
    RC = R // L
    NB = EPW // K
    mesh = plsc.VectorSubcoreMesh(core_axis_name="c", subcore_axis_name="s")

    slot_types = (
        pltpu.VMEM((K,), jnp.int32),
        pltpu.VMEM((K, L), jnp.float32),
        pltpu.VMEM((K, L), jnp.float32),
        pltpu.VMEM((K, L), jnp.float32),
        pltpu.VMEM((K, R), jnp.float32),
        pltpu.VMEM((K, R), jnp.float32),
        pltpu.SemaphoreType.DMA,
        pltpu.SemaphoreType.DMA,
        pltpu.SemaphoreType.DMA,
    )

    @functools.partial(
        pl.kernel,
        out_type=(jax.ShapeDtypeStruct((NC, N, R), jnp.float32),
                  jax.ShapeDtypeStruct((NC, N, L), jnp.float32)),
        mesh=mesh,
        compiler_params=pltpu.CompilerParams(use_tc_tiling_on_sc=False),
        scratch_types=(
            pltpu.VMEM_SHARED((N, R), jnp.float32),
            pltpu.VMEM_SHARED((N, L), jnp.float32),
            pltpu.VMEM((L,), jnp.float32),
            pltpu.VMEM((NB, K), jnp.int32),
        ) + slot_types + slot_types,
    )
    def sweep(src_hbm, dst2_hbm, h_hbm, elr_hbm, rle_hbm, m_hbm,
              out_hbm, den_hbm,
              out_sp, den_sp, m_v, dst_all, *slot_refs):
        ns = len(slot_types)
        slots = (slot_refs[:ns], slot_refs[ns:])
        cid = lax.axis_index("c")
        tid = lax.axis_index("s")
        wid = cid * NS + tid
        rbase = tid * ROWS_T

        ebase = wid * EPW
        pltpu.sync_copy(dst2_hbm.at[pl.ds(wid * NB, NB)], dst_all)

        zmsg = slots[0][5]
        zw = slots[0][3]

        def zo(i, c):
            for j in range(RC):
                zmsg[i, pl.ds(j * L, L)] = jnp.zeros((L,), jnp.float32)
            zw[i, :] = jnp.zeros((L,), jnp.float32)
            return c

        lax.fori_loop(0, K, zo, 0, unroll=4)
        for z in range(ROWS_T // K):
            pltpu.sync_copy(zmsg, out_sp.at[pl.ds(rbase + z * K, K)])
            pltpu.sync_copy(zw, den_sp.at[pl.ds(rbase + z * K, K)])
        rem = ROWS_T - (ROWS_T // K) * K
        pltpu.sync_copy(zmsg.at[pl.ds(0, rem)],
                        out_sp.at[pl.ds(rbase + ROWS_T - rem, rem)])
        pltpu.sync_copy(zw.at[pl.ds(0, rem)],
                        den_sp.at[pl.ds(rbase + ROWS_T - rem, rem)])

        @pl.when(tid == NS - 1)
        def _():
            tb = NS * ROWS_T
            pltpu.sync_copy(zmsg.at[pl.ds(0, TAIL)],
                            out_sp.at[pl.ds(tb, TAIL)])
            pltpu.sync_copy(zw.at[pl.ds(0, TAIL)],
                            den_sp.at[pl.ds(tb, TAIL)])

        pltpu.sync_copy(m_hbm, m_v)
        plsc.subcore_barrier()

        mvec = m_v[...]
        lane = lax.iota(jnp.int32, L)
        headmask = lane < 8

        def prefetch_src(s, b):
            src_v, _, _, _, _, _, semi, _, _ = slots[s]
            gb = pl.multiple_of(ebase + b * K, 8)
            pltpu.async_copy(src_hbm.at[pl.ds(gb, K)], src_v, semi)

        def issue_gathers(s, b):
            src_v, elrs_v, rled_v, _, rows_v, _, semi, semg, _ = slots[s]
            gb = pl.multiple_of(ebase + b * K, 8)
            pltpu.make_async_copy(src_hbm.at[pl.ds(gb, K)], src_v, semi).wait()
            pltpu.async_copy(elr_hbm.at[src_v], elrs_v, semg)
            pltpu.async_copy(rle_hbm.at[dst_all.at[b]], rled_v, semg)
            pltpu.async_copy(h_hbm.at[src_v], rows_v, semg)

        def wait_gathers(s, b):
            src_v, elrs_v, rled_v, _, rows_v, _, _, semg, _ = slots[s]
            pltpu.make_async_copy(elr_hbm.at[src_v], elrs_v, semg).wait()
            pltpu.make_async_copy(rle_hbm.at[dst_all.at[b]],
                                  rled_v, semg).wait()
            pltpu.make_async_copy(h_hbm.at[src_v], rows_v, semg).wait()

        def wait_scatters(s, b):
            _, _, _, w_v, _, msg_v, _, _, sems = slots[s]
            pltpu.make_async_copy(w_v, den_sp.at[dst_all.at[b]], sems).wait()
            pltpu.make_async_copy(msg_v, out_sp.at[dst_all.at[b]], sems).wait()

        def compute_and_scatter(s, b):
            _, elrs_v, rled_v, w_v, rows_v, msg_v, _, _, sems = slots[s]

            @plsc.parallel_loop(0, K, unroll=(4 if RC == 1 else 2))
            def _(k):
                e = elrs_v[k, :] + rled_v[k, :]
                e = jnp.where(e > 0, e, 0.2 * e)
                wrow = jnp.where(headmask, jnp.exp(e - mvec), 0.0)
                w_v[k, :] = wrow
                for j in range(RC):
                    msg_v[k, pl.ds(j * L, L)] = (
                        rows_v[k, pl.ds(j * L, L)] * _lane_bcast(wrow, j))

            pltpu.async_copy(w_v, den_sp.at[dst_all.at[b]], sems, add=True)
            pltpu.async_copy(msg_v, out_sp.at[dst_all.at[b]], sems, add=True)

        NPAIRS = NB // 2

        prefetch_src(0, 0)
        prefetch_src(1, 1)
        issue_gathers(0, 0)
        issue_gathers(1, 1)

        def pair(p, c):
            wait_gathers(0, 2 * p)

            @pl.when(2 * p + 2 < NB)
            def _():
                prefetch_src(0, 2 * p + 2)

            @pl.when(p > 0)
            def _():
                wait_scatters(0, 2 * p - 2)

            compute_and_scatter(0, 2 * p)

            @pl.when(2 * p + 2 < NB)
            def _():
                issue_gathers(0, 2 * p + 2)

            wait_gathers(1, 2 * p + 1)

            @pl.when(2 * p + 3 < NB)
            def _():
                prefetch_src(1, 2 * p + 3)

            @pl.when(p > 0)
            def _():
                wait_scatters(1, 2 * p - 1)

            compute_and_scatter(1, 2 * p + 1)

            @pl.when(2 * p + 3 < NB)
            def _():
                issue_gathers(1, 2 * p + 3)

            return c

        lax.fori_loop(0, NPAIRS, pair, 0)

        if NB % 2:
            wait_gathers(0, NB - 1)
            wait_scatters(0, NB - 3)
            compute_and_scatter(0, NB - 1)
            wait_scatters(0, NB - 1)
            wait_scatters(1, NB - 2)
        else:
            wait_scatters(0, NB - 2)
            wait_scatters(1, NB - 1)

        plsc.subcore_barrier()
        pltpu.sync_copy(out_sp.at[pl.ds(rbase, ROWS_T)],
                        out_hbm.at[cid, pl.ds(rbase, ROWS_T)])
        pltpu.sync_copy(den_sp.at[pl.ds(rbase, ROWS_T)],
                        den_hbm.at[cid, pl.ds(rbase, ROWS_T)])

        @pl.when(tid == NS - 1)
        def _():
            tb = NS * ROWS_T
            pltpu.sync_copy(out_sp.at[pl.ds(tb, TAIL)],
                            out_hbm.at[cid, pl.ds(tb, TAIL)])
            pltpu.sync_copy(den_sp.at[pl.ds(tb, TAIL)],
                            den_hbm.at[cid, pl.ds(tb, TAIL)])

    return sweep


_edge_sweep_128 = _make_edge_sweep(D, K1)
_edge_sweep_16 = _make_edge_sweep(L, K2)


def _pack(al, ar):
    H, Fo = al.shape
    eye = jnp.eye(H, 8, dtype=jnp.float32)
    a_el = (al[:, :, None] * eye[:, None, :]).reshape(H * Fo, 8)
    a_er = (ar[:, :, None] * eye[:, None, :]).reshape(H * Fo, 8)
    return (jnp.concatenate([a_el, a_er], 1).astype(jnp.float32),
            jnp.concatenate([a_er, a_el], 1).astype(jnp.float32))


def _mtile(mx, H):
    m = mx[0]
    s = m[:8] + m[8:]
    s = jnp.where(s > 0, s, 0.2 * s)
    head = jnp.where(jnp.arange(8) < H, s, 1e30)
    return jnp.concatenate([head, jnp.full((8,), 1e30, jnp.float32)])


def kernel(feats, edge_index, W1, al1, ar1, b1, W2, al2, ar2, b2):
    pe1, pr1 = _pack(al1, ar1)
    pe2, pr2 = _pack(al2, ar2)
    q = (jnp.eye(L, 8, dtype=jnp.float32)[:, :, None]
         * jnp.ones((1, 1, L), jnp.float32)).reshape(L, D)
    q2 = jnp.zeros((L, L), jnp.float32).at[0, :].set(1.0)

    src = edge_index[0]
    dst = edge_index[1]
    h1, elr1, rle1, mx1 = _tc_project(feats, W1, pe1, pr1)
    m1 = _mtile(mx1, 8)
    out1p, den1p = _edge_sweep_128(src, dst.reshape(E // K1, K1),
                                   h1, elr1, rle1, m1)

    h2, elr2, rle2, mx2 = _tc_combine_project(
        out1p, den1p, W2, q, b1.reshape(1, D), pe2, pr2)
    m2 = _mtile(mx2, 1)
    out2p, den2p = _edge_sweep_16(src, dst.reshape(E // K2, K2),
                                  h2, elr2, rle2, m2)

    return _tc_finish(out2p, den2p, q2, b2.reshape(1, L))


# speedup vs baseline: 3.0958x; 1.0131x over previous
import functools

import jax
import jax.numpy as jnp
from jax import lax
from jax.experimental import pallas as pl
from jax.experimental.pallas import tpu as pltpu
from jax.experimental.pallas import tpu_sc as plsc

N = 10000
E = 320000
D = 128
NC, NS, L = 2, 16, 16
NW = NC * NS
EPW = E // NW
K1 = 40
K2 = 80
ROWS_T = 624
TAIL = N - NS * ROWS_T
BN = 1000


def _tc1_body(x_ref, w_ref, pe_ref, pr_ref, hx_ref, rle_ref, mx_ref):
    i = pl.program_id(0)
    h = jnp.dot(x_ref[...], w_ref[...], preferred_element_type=jnp.float32)
    t = jnp.dot(h, pe_ref[...], preferred_element_type=jnp.float32)
    hx_ref[...] = jnp.concatenate([h, t], axis=1)
    rle_ref[...] = jnp.dot(h, pr_ref[...], preferred_element_type=jnp.float32)

    @pl.when(i == 0)
    def _():
        mx_ref[...] = jnp.full((1, L), -1e30, jnp.float32)

    mx_ref[...] = jnp.maximum(mx_ref[...], jnp.max(t, axis=0, keepdims=True))


def _tc_project(x, W, Pe, Pr):
    n, d = x.shape
    r = W.shape[1]
    return pl.pallas_call(
        _tc1_body,
        grid=(n // BN,),
        in_specs=[pl.BlockSpec((BN, d), lambda i: (i, 0)),
                  pl.BlockSpec((d, r), lambda i: (0, 0)),
                  pl.BlockSpec((r, L), lambda i: (0, 0)),
                  pl.BlockSpec((r, L), lambda i: (0, 0))],
        out_specs=[pl.BlockSpec((BN, r + L), lambda i: (i, 0)),
                   pl.BlockSpec((BN, L), lambda i: (i, 0)),
                   pl.BlockSpec((1, L), lambda i: (0, 0))],
        out_shape=[jax.ShapeDtypeStruct((n, r + L), jnp.float32),
                   jax.ShapeDtypeStruct((n, L), jnp.float32),
                   jax.ShapeDtypeStruct((1, L), jnp.float32)],
    )(x, W, Pe, Pr)


def _tc2_body(p_ref, w_ref, q_ref, b_ref, pe_ref, pr_ref,
              hx2_ref, rle_ref, mx_ref):
    i = pl.program_id(0)
    acc = p_ref[0] + p_ref[1]
    num = acc[:, :D]
    den = acc[:, D:]
    den128 = jnp.dot(den, q_ref[...], preferred_element_type=jnp.float32)
    x2 = num / (den128 + 1e-9) + b_ref[...]
    x2 = jnp.maximum(x2, 0.0)
    h2 = jnp.dot(x2, w_ref[...], preferred_element_type=jnp.float32)
    t = jnp.dot(h2, pe_ref[...], preferred_element_type=jnp.float32)
    hx2_ref[...] = jnp.concatenate([h2, t], axis=1)
    rle_ref[...] = jnp.dot(h2, pr_ref[...], preferred_element_type=jnp.float32)

    @pl.when(i == 0)
    def _():
        mx_ref[...] = jnp.full((1, L), -1e30, jnp.float32)

    mx_ref[...] = jnp.maximum(mx_ref[...], jnp.max(t, axis=0, keepdims=True))


def _tc_combine_project(outp, W2, Q, b1row, Pe, Pr):
    return pl.pallas_call(
        _tc2_body,
        grid=(N // BN,),
        in_specs=[pl.BlockSpec((NC, BN, D + L), lambda i: (0, i, 0)),
                  pl.BlockSpec((D, L), lambda i: (0, 0)),
                  pl.BlockSpec((L, D), lambda i: (0, 0)),
                  pl.BlockSpec((1, D), lambda i: (0, 0)),
                  pl.BlockSpec((L, L), lambda i: (0, 0)),
                  pl.BlockSpec((L, L), lambda i: (0, 0))],
        out_specs=[pl.BlockSpec((BN, 2 * L), lambda i: (i, 0)),
                   pl.BlockSpec((BN, L), lambda i: (i, 0)),
                   pl.BlockSpec((1, L), lambda i: (0, 0))],
        out_shape=[jax.ShapeDtypeStruct((N, 2 * L), jnp.float32),
                   jax.ShapeDtypeStruct((N, L), jnp.float32),
                   jax.ShapeDtypeStruct((1, L), jnp.float32)],
    )(outp, W2, Q, b1row, Pe, Pr)


def _tc3_body(p_ref, q2_ref, b_ref, o_ref):
    acc = p_ref[0] + p_ref[1]
    num = acc[:, :L]
    den = acc[:, L:]
    den16 = jnp.dot(den, q2_ref[...], preferred_element_type=jnp.float32)
    o_ref[...] = num / (den16 + 1e-9) + b_ref[...]


def _tc_finish(outp, Q2, b2row):
    return pl.pallas_call(
        _tc3_body,
        grid=(N // BN,),
        in_specs=[pl.BlockSpec((NC, BN, 2 * L), lambda i: (0, i, 0)),
                  pl.BlockSpec((L, L), lambda i: (0, 0)),
                  pl.BlockSpec((1, L), lambda i: (0, 0))],
        out_specs=pl.BlockSpec((BN, L), lambda i: (i, 0)),
        out_shape=jax.ShapeDtypeStruct((N, L), jnp.float32),
    )(outp, Q2, b2row)


_BCAST_DNUMS = lax.GatherDimensionNumbers(
    offset_dims=(), collapsed_slice_dims=(0,), start_index_map=(0,))


def _lane_bcast(v, j):
    idx = jnp.full((L, 1), j, jnp.int32)
    return lax.gather(v, idx, _BCAST_DNUMS, (1,),
                      mode=lax.GatherScatterMode.PROMISE_IN_BOUNDS)


def _make_edge_sweep(R, K):
    RC = R // L
    RW = R + L
    NB = EPW // K
    mesh = plsc.VectorSubcoreMesh(core_axis_name="c", subcore_axis_name="s")

    slot_types = (
        pltpu.VMEM((K,), jnp.int32),
        pltpu.VMEM((K, L), jnp.float32),
        pltpu.VMEM((K, RW), jnp.float32),
        pltpu.VMEM((K, RW), jnp.float32),
        pltpu.SemaphoreType.DMA,
        pltpu.SemaphoreType.DMA,
        pltpu.SemaphoreType.DMA,
    )

    @functools.partial(
        pl.kernel,
        out_type=jax.ShapeDtypeStruct((NC, N, RW), jnp.float32),
        mesh=mesh,
        compiler_params=pltpu.CompilerParams(use_tc_tiling_on_sc=False),
        scratch_types=(
            pltpu.VMEM_SHARED((N, RW), jnp.float32),
            pltpu.VMEM((L,), jnp.float32),
            pltpu.VMEM((NB, K), jnp.int32),
        ) + slot_types + slot_types,
    )
    def sweep(src_hbm, dst2_hbm, hx_hbm, rle_hbm, m_hbm,
              out_hbm,
              out_sp, m_v, dst_all, *slot_refs):
        ns = len(slot_types)
        slots = (slot_refs[:ns], slot_refs[ns:])
        cid = lax.axis_index("c")
        tid = lax.axis_index("s")
        wid = cid * NS + tid
        rbase = tid * ROWS_T

        ebase = wid * EPW
        pltpu.sync_copy(dst2_hbm.at[pl.ds(wid * NB, NB)], dst_all)

        zmsg = slots[0][3]

        def zo(i, c):
            for j in range(RW // L):
                zmsg[i, pl.ds(j * L, L)] = jnp.zeros((L,), jnp.float32)
            return c

        lax.fori_loop(0, K, zo, 0, unroll=4)
        for z in range(ROWS_T // K):
            pltpu.sync_copy(zmsg, out_sp.at[pl.ds(rbase + z * K, K)])
        rem = ROWS_T - (ROWS_T // K) * K
        pltpu.sync_copy(zmsg.at[pl.ds(0, rem)],
                        out_sp.at[pl.ds(rbase + ROWS_T - rem, rem)])

        @pl.when(tid == NS - 1)
        def _():
            tb = NS * ROWS_T
            pltpu.sync_copy(zmsg.at[pl.ds(0, TAIL)],
                            out_sp.at[pl.ds(tb, TAIL)])

        pltpu.sync_copy(m_hbm, m_v)
        plsc.subcore_barrier()

        mvec = m_v[...]
        lane = lax.iota(jnp.int32, L)
        headmask = lane < 8

        def prefetch_src(s, b):
            src_v, _, _, _, semi, _, _ = slots[s]
            gb = pl.multiple_of(ebase + b * K, 8)
            pltpu.async_copy(src_hbm.at[pl.ds(gb, K)], src_v, semi)

        def issue_gathers(s, b):
            src_v, rled_v, rows_v, _, semi, semg, _ = slots[s]
            gb = pl.multiple_of(ebase + b * K, 8)
            pltpu.make_async_copy(src_hbm.at[pl.ds(gb, K)], src_v, semi).wait()
            pltpu.async_copy(rle_hbm.at[dst_all.at[b]], rled_v, semg)
            pltpu.async_copy(hx_hbm.at[src_v], rows_v, semg)

        def wait_gathers(s, b):
            src_v, rled_v, rows_v, _, _, semg, _ = slots[s]
            pltpu.make_async_copy(rle_hbm.at[dst_all.at[b]],
                                  rled_v, semg).wait()
            pltpu.make_async_copy(hx_hbm.at[src_v], rows_v, semg).wait()

        def wait_scatters(s, b):
            _, _, _, msg_v, _, _, sems = slots[s]
            pltpu.make_async_copy(msg_v, out_sp.at[dst_all.at[b]], sems).wait()

        def compute_and_scatter(s, b):
            _, rled_v, rows_v, msg_v, _, _, sems = slots[s]

            @plsc.parallel_loop(0, K, unroll=(4 if RC == 1 else 2))
            def _(k):
                e = rows_v[k, pl.ds(R, L)] + rled_v[k, :]
                e = jnp.where(e > 0, e, 0.2 * e)
                wrow = jnp.where(headmask, jnp.exp(e - mvec), 0.0)
                msg_v[k, pl.ds(R, L)] = wrow
                for j in range(RC):
                    msg_v[k, pl.ds(j * L, L)] = (
                        rows_v[k, pl.ds(j * L, L)] * _lane_bcast(wrow, j))

            pltpu.async_copy(msg_v, out_sp.at[dst_all.at[b]], sems, add=True)

        NPAIRS = NB // 2

        prefetch_src(0, 0)
        prefetch_src(1, 1)
        issue_gathers(0, 0)
        issue_gathers(1, 1)

        def pair(p, c):
            wait_gathers(0, 2 * p)

            @pl.when(2 * p + 2 < NB)
            def _():
                prefetch_src(0, 2 * p + 2)

            @pl.when(p > 0)
            def _():
                wait_scatters(0, 2 * p - 2)

            compute_and_scatter(0, 2 * p)

            @pl.when(2 * p + 2 < NB)
            def _():
                issue_gathers(0, 2 * p + 2)

            wait_gathers(1, 2 * p + 1)

            @pl.when(2 * p + 3 < NB)
            def _():
                prefetch_src(1, 2 * p + 3)

            @pl.when(p > 0)
            def _():
                wait_scatters(1, 2 * p - 1)

            compute_and_scatter(1, 2 * p + 1)

            @pl.when(2 * p + 3 < NB)
            def _():
                issue_gathers(1, 2 * p + 3)

            return c

        lax.fori_loop(0, NPAIRS, pair, 0)

        if NB % 2:
            wait_gathers(0, NB - 1)
            wait_scatters(0, NB - 3)
            compute_and_scatter(0, NB - 1)
            wait_scatters(0, NB - 1)
            wait_scatters(1, NB - 2)
        else:
            wait_scatters(0, NB - 2)
            wait_scatters(1, NB - 1)

        plsc.subcore_barrier()
        pltpu.sync_copy(out_sp.at[pl.ds(rbase, ROWS_T)],
                        out_hbm.at[cid, pl.ds(rbase, ROWS_T)])

        @pl.when(tid == NS - 1)
        def _():
            tb = NS * ROWS_T
            pltpu.sync_copy(out_sp.at[pl.ds(tb, TAIL)],
                            out_hbm.at[cid, pl.ds(tb, TAIL)])

    return sweep


_edge_sweep_128 = _make_edge_sweep(D, K1)
_edge_sweep_16 = _make_edge_sweep(L, K2)


def _pack(al, ar):
    H, Fo = al.shape
    eye = jnp.eye(H, 8, dtype=jnp.float32)
    a_el = (al[:, :, None] * eye[:, None, :]).reshape(H * Fo, 8)
    a_er = (ar[:, :, None] * eye[:, None, :]).reshape(H * Fo, 8)
    return (jnp.concatenate([a_el, a_er], 1).astype(jnp.float32),
            jnp.concatenate([a_er, a_el], 1).astype(jnp.float32))


def _mtile(mx, H):
    m = mx[0]
    s = m[:8] + m[8:]
    s = jnp.where(s > 0, s, 0.2 * s)
    head = jnp.where(jnp.arange(8) < H, s, 1e30)
    return jnp.concatenate([head, jnp.full((8,), 1e30, jnp.float32)])


def kernel(feats, edge_index, W1, al1, ar1, b1, W2, al2, ar2, b2):
    pe1, pr1 = _pack(al1, ar1)
    pe2, pr2 = _pack(al2, ar2)
    q = (jnp.eye(L, 8, dtype=jnp.float32)[:, :, None]
         * jnp.ones((1, 1, L), jnp.float32)).reshape(L, D)
    q2 = jnp.zeros((L, L), jnp.float32).at[0, :].set(1.0)

    src = edge_index[0]
    dst = edge_index[1]
    hx1, rle1, mx1 = _tc_project(feats, W1, pe1, pr1)
    m1 = _mtile(mx1, 8)
    out1p = _edge_sweep_128(src, dst.reshape(E // K1, K1), hx1, rle1, m1)

    hx2, rle2, mx2 = _tc_combine_project(
        out1p, W2, q, b1.reshape(1, D), pe2, pr2)
    m2 = _mtile(mx2, 1)
    out2p = _edge_sweep_16(src, dst.reshape(E // K2, K2), hx2, rle2, m2)

    return _tc_finish(out2p, q2, b2.reshape(1, L))


# speedup vs baseline: 3.0998x; 1.0013x over previous
import functools

import jax
import jax.numpy as jnp
from jax import lax
from jax.experimental import pallas as pl
from jax.experimental.pallas import tpu as pltpu
from jax.experimental.pallas import tpu_sc as plsc

N = 10000
E = 320000
D = 128
NC, NS, L = 2, 16, 16
NW = NC * NS
EPW = E // NW
K1 = 40
K2 = 80
ROWS_T = 624
TAIL = N - NS * ROWS_T
BN = 1000


def _tc1_body(x_ref, w_ref, pe_ref, pr_ref, hx_ref, rle_ref, mx_ref):
    i = pl.program_id(0)
    h = jnp.dot(x_ref[...], w_ref[...], preferred_element_type=jnp.float32)
    t = jnp.dot(h, pe_ref[...], preferred_element_type=jnp.float32)
    hx_ref[...] = jnp.concatenate([h, t], axis=1)
    rle_ref[...] = jnp.dot(h, pr_ref[...], preferred_element_type=jnp.float32)

    @pl.when(i == 0)
    def _():
        mx_ref[...] = jnp.full((1, L), -1e30, jnp.float32)

    mx_ref[...] = jnp.maximum(mx_ref[...], jnp.max(t, axis=0, keepdims=True))


def _tc_project(x, W, Pe, Pr):
    n, d = x.shape
    r = W.shape[1]
    return pl.pallas_call(
        _tc1_body,
        grid=(n // BN,),
        in_specs=[pl.BlockSpec((BN, d), lambda i: (i, 0)),
                  pl.BlockSpec((d, r), lambda i: (0, 0)),
                  pl.BlockSpec((r, L), lambda i: (0, 0)),
                  pl.BlockSpec((r, L), lambda i: (0, 0))],
        out_specs=[pl.BlockSpec((BN, r + L), lambda i: (i, 0)),
                   pl.BlockSpec((BN, L), lambda i: (i, 0)),
                   pl.BlockSpec((1, L), lambda i: (0, 0))],
        out_shape=[jax.ShapeDtypeStruct((n, r + L), jnp.float32),
                   jax.ShapeDtypeStruct((n, L), jnp.float32),
                   jax.ShapeDtypeStruct((1, L), jnp.float32)],
    )(x, W, Pe, Pr)


def _tc2_body(p_ref, w_ref, q_ref, b_ref, pe_ref, pr_ref,
              hx2_ref, rle_ref, mx_ref):
    i = pl.program_id(0)
    acc = p_ref[0] + p_ref[1]
    num = acc[:, :D]
    den = acc[:, D:]
    den128 = jnp.dot(den, q_ref[...], preferred_element_type=jnp.float32)
    x2 = num / (den128 + 1e-9) + b_ref[...]
    x2 = jnp.maximum(x2, 0.0)
    h2 = jnp.dot(x2, w_ref[...], preferred_element_type=jnp.float32)
    t = jnp.dot(h2, pe_ref[...], preferred_element_type=jnp.float32)
    hx2_ref[...] = jnp.concatenate([h2, t], axis=1)
    rle_ref[...] = jnp.dot(h2, pr_ref[...], preferred_element_type=jnp.float32)

    @pl.when(i == 0)
    def _():
        mx_ref[...] = jnp.full((1, L), -1e30, jnp.float32)

    mx_ref[...] = jnp.maximum(mx_ref[...], jnp.max(t, axis=0, keepdims=True))


def _tc_combine_project(outp, W2, Q, b1row, Pe, Pr):
    return pl.pallas_call(
        _tc2_body,
        grid=(N // BN,),
        in_specs=[pl.BlockSpec((NC, BN, D + L), lambda i: (0, i, 0)),
                  pl.BlockSpec((D, L), lambda i: (0, 0)),
                  pl.BlockSpec((L, D), lambda i: (0, 0)),
                  pl.BlockSpec((1, D), lambda i: (0, 0)),
                  pl.BlockSpec((L, L), lambda i: (0, 0)),
                  pl.BlockSpec((L, L), lambda i: (0, 0))],
        out_specs=[pl.BlockSpec((BN, 2 * L), lambda i: (i, 0)),
                   pl.BlockSpec((BN, L), lambda i: (i, 0)),
                   pl.BlockSpec((1, L), lambda i: (0, 0))],
        out_shape=[jax.ShapeDtypeStruct((N, 2 * L), jnp.float32),
                   jax.ShapeDtypeStruct((N, L), jnp.float32),
                   jax.ShapeDtypeStruct((1, L), jnp.float32)],
    )(outp, W2, Q, b1row, Pe, Pr)


def _tc3_body(p_ref, q2_ref, b_ref, o_ref):
    acc = p_ref[0] + p_ref[1]
    num = acc[:, :L]
    den = acc[:, L:]
    den16 = jnp.dot(den, q2_ref[...], preferred_element_type=jnp.float32)
    o_ref[...] = num / (den16 + 1e-9) + b_ref[...]


def _tc_finish(outp, Q2, b2row):
    return pl.pallas_call(
        _tc3_body,
        grid=(N // BN,),
        in_specs=[pl.BlockSpec((NC, BN, 2 * L), lambda i: (0, i, 0)),
                  pl.BlockSpec((L, L), lambda i: (0, 0)),
                  pl.BlockSpec((1, L), lambda i: (0, 0))],
        out_specs=pl.BlockSpec((BN, L), lambda i: (i, 0)),
        out_shape=jax.ShapeDtypeStruct((N, L), jnp.float32),
    )(outp, Q2, b2row)


_BCAST_DNUMS = lax.GatherDimensionNumbers(
    offset_dims=(), collapsed_slice_dims=(0,), start_index_map=(0,))


def _lane_bcast(v, j):
    idx = jnp.full((L, 1), j, jnp.int32)
    return lax.gather(v, idx, _BCAST_DNUMS, (1,),
                      mode=lax.GatherScatterMode.PROMISE_IN_BOUNDS)


def _make_edge_sweep(R, K):
    RC = R // L
    RW = R + L
    NB = EPW // K
    mesh = plsc.VectorSubcoreMesh(core_axis_name="c", subcore_axis_name="s")

    slot_types = (
        pltpu.VMEM((K,), jnp.int32),
        pltpu.VMEM((K, L), jnp.float32),
        pltpu.VMEM((K, RW), jnp.float32),
        pltpu.VMEM((K, RW), jnp.float32),
        pltpu.SemaphoreType.DMA,
        pltpu.SemaphoreType.DMA,
        pltpu.SemaphoreType.DMA,
    )

    @functools.partial(
        pl.kernel,
        out_type=jax.ShapeDtypeStruct((NC, N, RW), jnp.float32),
        mesh=mesh,
        compiler_params=pltpu.CompilerParams(use_tc_tiling_on_sc=False),
        scratch_types=(
            pltpu.VMEM_SHARED((N, RW), jnp.float32),
            pltpu.VMEM((L,), jnp.float32),
            pltpu.VMEM((NB, K), jnp.int32),
        ) + slot_types + slot_types,
    )
    def sweep(src_hbm, dst2_hbm, hx_hbm, rle_hbm, m_hbm,
              out_hbm,
              out_sp, m_v, dst_all, *slot_refs):
        ns = len(slot_types)
        slots = (slot_refs[:ns], slot_refs[ns:])
        cid = lax.axis_index("c")
        tid = lax.axis_index("s")
        wid = cid * NS + tid
        rbase = tid * ROWS_T

        ebase = wid * EPW
        pltpu.sync_copy(dst2_hbm.at[pl.ds(wid * NB, NB)], dst_all)

        zmsg = slots[0][3]

        def zo(i, c):
            for j in range(RW // L):
                zmsg[i, pl.ds(j * L, L)] = jnp.zeros((L,), jnp.float32)
            return c

        lax.fori_loop(0, K, zo, 0, unroll=4)
        for z in range(ROWS_T // K):
            pltpu.sync_copy(zmsg, out_sp.at[pl.ds(rbase + z * K, K)])
        rem = ROWS_T - (ROWS_T // K) * K
        pltpu.sync_copy(zmsg.at[pl.ds(0, rem)],
                        out_sp.at[pl.ds(rbase + ROWS_T - rem, rem)])

        @pl.when(tid == NS - 1)
        def _():
            tb = NS * ROWS_T
            pltpu.sync_copy(zmsg.at[pl.ds(0, TAIL)],
                            out_sp.at[pl.ds(tb, TAIL)])

        pltpu.sync_copy(m_hbm, m_v)
        plsc.subcore_barrier()

        mvec = m_v[...]
        lane = lax.iota(jnp.int32, L)
        headmask = lane < 8

        def prefetch_src(s, b):
            src_v, _, _, _, semi, _, _ = slots[s]
            gb = pl.multiple_of(ebase + b * K, 8)
            pltpu.async_copy(src_hbm.at[pl.ds(gb, K)], src_v, semi)

        def issue_gathers(s, b):
            src_v, rled_v, rows_v, _, semi, semg, _ = slots[s]
            gb = pl.multiple_of(ebase + b * K, 8)
            pltpu.make_async_copy(src_hbm.at[pl.ds(gb, K)], src_v, semi).wait()
            pltpu.async_copy(rle_hbm.at[dst_all.at[b]], rled_v, semg)
            pltpu.async_copy(hx_hbm.at[src_v], rows_v, semg)

        def wait_gathers(s, b):
            src_v, rled_v, rows_v, _, _, semg, _ = slots[s]
            pltpu.make_async_copy(rle_hbm.at[dst_all.at[b]],
                                  rled_v, semg).wait()
            pltpu.make_async_copy(hx_hbm.at[src_v], rows_v, semg).wait()

        def wait_scatters(s, b):
            _, _, _, msg_v, _, _, sems = slots[s]
            pltpu.make_async_copy(msg_v, out_sp.at[dst_all.at[b]], sems).wait()

        def compute_and_scatter(s, b):
            _, rled_v, rows_v, msg_v, _, _, sems = slots[s]

            @plsc.parallel_loop(0, K, unroll=4)
            def _(k):
                e = rows_v[k, pl.ds(R, L)] + rled_v[k, :]
                e = jnp.where(e > 0, e, 0.2 * e)
                wrow = jnp.where(headmask, jnp.exp(e - mvec), 0.0)
                msg_v[k, pl.ds(R, L)] = wrow
                for j in range(RC):
                    msg_v[k, pl.ds(j * L, L)] = (
                        rows_v[k, pl.ds(j * L, L)] * _lane_bcast(wrow, j))

            pltpu.async_copy(msg_v, out_sp.at[dst_all.at[b]], sems, add=True)

        NPAIRS = NB // 2

        prefetch_src(0, 0)
        prefetch_src(1, 1)
        issue_gathers(0, 0)
        issue_gathers(1, 1)

        def pair(p, c):
            wait_gathers(0, 2 * p)

            @pl.when(2 * p + 2 < NB)
            def _():
                prefetch_src(0, 2 * p + 2)

            @pl.when(p > 0)
            def _():
                wait_scatters(0, 2 * p - 2)

            compute_and_scatter(0, 2 * p)

            @pl.when(2 * p + 2 < NB)
            def _():
                issue_gathers(0, 2 * p + 2)

            wait_gathers(1, 2 * p + 1)

            @pl.when(2 * p + 3 < NB)
            def _():
                prefetch_src(1, 2 * p + 3)

            @pl.when(p > 0)
            def _():
                wait_scatters(1, 2 * p - 1)

            compute_and_scatter(1, 2 * p + 1)

            @pl.when(2 * p + 3 < NB)
            def _():
                issue_gathers(1, 2 * p + 3)

            return c

        lax.fori_loop(0, NPAIRS, pair, 0)

        if NB % 2:
            wait_gathers(0, NB - 1)
            wait_scatters(0, NB - 3)
            compute_and_scatter(0, NB - 1)
            wait_scatters(0, NB - 1)
            wait_scatters(1, NB - 2)
        else:
            wait_scatters(0, NB - 2)
            wait_scatters(1, NB - 1)

        plsc.subcore_barrier()
        pltpu.sync_copy(out_sp.at[pl.ds(rbase, ROWS_T)],
                        out_hbm.at[cid, pl.ds(rbase, ROWS_T)])

        @pl.when(tid == NS - 1)
        def _():
            tb = NS * ROWS_T
            pltpu.sync_copy(out_sp.at[pl.ds(tb, TAIL)],
                            out_hbm.at[cid, pl.ds(tb, TAIL)])

    return sweep


_edge_sweep_128 = _make_edge_sweep(D, K1)
_edge_sweep_16 = _make_edge_sweep(L, K2)


def _pack(al, ar):
    H, Fo = al.shape
    eye = jnp.eye(H, 8, dtype=jnp.float32)
    a_el = (al[:, :, None] * eye[:, None, :]).reshape(H * Fo, 8)
    a_er = (ar[:, :, None] * eye[:, None, :]).reshape(H * Fo, 8)
    return (jnp.concatenate([a_el, a_er], 1).astype(jnp.float32),
            jnp.concatenate([a_er, a_el], 1).astype(jnp.float32))


def _mtile(mx, H):
    m = mx[0]
    s = m[:8] + m[8:]
    s = jnp.where(s > 0, s, 0.2 * s)
    head = jnp.where(jnp.arange(8) < H, s, 1e30)
    return jnp.concatenate([head, jnp.full((8,), 1e30, jnp.float32)])


def kernel(feats, edge_index, W1, al1, ar1, b1, W2, al2, ar2, b2):
    pe1, pr1 = _pack(al1, ar1)
    pe2, pr2 = _pack(al2, ar2)
    q = (jnp.eye(L, 8, dtype=jnp.float32)[:, :, None]
         * jnp.ones((1, 1, L), jnp.float32)).reshape(L, D)
    q2 = jnp.zeros((L, L), jnp.float32).at[0, :].set(1.0)

    src = edge_index[0]
    dst = edge_index[1]
    hx1, rle1, mx1 = _tc_project(feats, W1, pe1, pr1)
    m1 = _mtile(mx1, 8)
    out1p = _edge_sweep_128(src, dst.reshape(E // K1, K1), hx1, rle1, m1)

    hx2, rle2, mx2 = _tc_combine_project(
        out1p, W2, q, b1.reshape(1, D), pe2, pr2)
    m2 = _mtile(mx2, 1)
    out2p = _edge_sweep_16(src, dst.reshape(E // K2, K2), hx2, rle2, m2)

    return _tc_finish(out2p, q2, b2.reshape(1, L))


# speedup vs baseline: 3.1162x; 1.0053x over previous
import functools

import jax
import jax.numpy as jnp
from jax import lax
from jax.experimental import pallas as pl
from jax.experimental.pallas import tpu as pltpu
from jax.experimental.pallas import tpu_sc as plsc

N = 10000
E = 320000
D = 128
NC, NS, L = 2, 16, 16
NW = NC * NS
EPW = E // NW
K1 = 40
K2 = 80
ROWS_T = 624
TAIL = N - NS * ROWS_T
BN = 1000


def _tc1_body(x_ref, w_ref, pe_ref, pr_ref, hx_ref, rle_ref, mx_ref):
    i = pl.program_id(0)
    h = jnp.dot(x_ref[...], w_ref[...], preferred_element_type=jnp.float32)
    t = jnp.dot(h, pe_ref[...], preferred_element_type=jnp.float32)
    hx_ref[...] = jnp.concatenate([h, t], axis=1)
    rle_ref[...] = jnp.dot(h, pr_ref[...], preferred_element_type=jnp.float32)

    @pl.when(i == 0)
    def _():
        mx_ref[...] = jnp.full((1, L), -1e30, jnp.float32)

    mx_ref[...] = jnp.maximum(mx_ref[...], jnp.max(t, axis=0, keepdims=True))


def _tc_project(x, W, Pe, Pr):
    n, d = x.shape
    r = W.shape[1]
    return pl.pallas_call(
        _tc1_body,
        grid=(n // BN,),
        in_specs=[pl.BlockSpec((BN, d), lambda i: (i, 0)),
                  pl.BlockSpec((d, r), lambda i: (0, 0)),
                  pl.BlockSpec((r, L), lambda i: (0, 0)),
                  pl.BlockSpec((r, L), lambda i: (0, 0))],
        out_specs=[pl.BlockSpec((BN, r + L), lambda i: (i, 0)),
                   pl.BlockSpec((BN, L), lambda i: (i, 0)),
                   pl.BlockSpec((1, L), lambda i: (0, 0))],
        out_shape=[jax.ShapeDtypeStruct((n, r + L), jnp.float32),
                   jax.ShapeDtypeStruct((n, L), jnp.float32),
                   jax.ShapeDtypeStruct((1, L), jnp.float32)],
    )(x, W, Pe, Pr)


def _tc2_body(p_ref, w_ref, q_ref, b_ref, pe_ref, pr_ref,
              hx2_ref, rle_ref, mx_ref):
    i = pl.program_id(0)
    acc = p_ref[0] + p_ref[1]
    num = acc[:, :D]
    den = acc[:, D:]
    den128 = jnp.dot(den, q_ref[...], preferred_element_type=jnp.float32)
    x2 = num / (den128 + 1e-9) + b_ref[...]
    x2 = jnp.maximum(x2, 0.0)
    h2 = jnp.dot(x2, w_ref[...], preferred_element_type=jnp.float32)
    t = jnp.dot(h2, pe_ref[...], preferred_element_type=jnp.float32)
    hx2_ref[...] = jnp.concatenate([h2, t], axis=1)
    rle_ref[...] = jnp.dot(h2, pr_ref[...], preferred_element_type=jnp.float32)

    @pl.when(i == 0)
    def _():
        mx_ref[...] = jnp.full((1, L), -1e30, jnp.float32)

    mx_ref[...] = jnp.maximum(mx_ref[...], jnp.max(t, axis=0, keepdims=True))


def _tc_combine_project(outp, W2, Q, b1row, Pe, Pr):
    return pl.pallas_call(
        _tc2_body,
        grid=(N // BN,),
        in_specs=[pl.BlockSpec((NC, BN, D + L), lambda i: (0, i, 0)),
                  pl.BlockSpec((D, L), lambda i: (0, 0)),
                  pl.BlockSpec((L, D), lambda i: (0, 0)),
                  pl.BlockSpec((1, D), lambda i: (0, 0)),
                  pl.BlockSpec((L, L), lambda i: (0, 0)),
                  pl.BlockSpec((L, L), lambda i: (0, 0))],
        out_specs=[pl.BlockSpec((BN, 2 * L), lambda i: (i, 0)),
                   pl.BlockSpec((BN, L), lambda i: (i, 0)),
                   pl.BlockSpec((1, L), lambda i: (0, 0))],
        out_shape=[jax.ShapeDtypeStruct((N, 2 * L), jnp.float32),
                   jax.ShapeDtypeStruct((N, L), jnp.float32),
                   jax.ShapeDtypeStruct((1, L), jnp.float32)],
    )(outp, W2, Q, b1row, Pe, Pr)


def _tc3_body(p_ref, q2_ref, b_ref, o_ref):
    acc = p_ref[0] + p_ref[1]
    num = acc[:, :L]
    den = acc[:, L:]
    den16 = jnp.dot(den, q2_ref[...], preferred_element_type=jnp.float32)
    o_ref[...] = num / (den16 + 1e-9) + b_ref[...]


def _tc_finish(outp, Q2, b2row):
    return pl.pallas_call(
        _tc3_body,
        grid=(N // BN,),
        in_specs=[pl.BlockSpec((NC, BN, 2 * L), lambda i: (0, i, 0)),
                  pl.BlockSpec((L, L), lambda i: (0, 0)),
                  pl.BlockSpec((1, L), lambda i: (0, 0))],
        out_specs=pl.BlockSpec((BN, L), lambda i: (i, 0)),
        out_shape=jax.ShapeDtypeStruct((N, L), jnp.float32),
    )(outp, Q2, b2row)


_BCAST_DNUMS = lax.GatherDimensionNumbers(
    offset_dims=(), collapsed_slice_dims=(0,), start_index_map=(0,))


def _lane_bcast(v, j):
    idx = jnp.full((L, 1), j, jnp.int32)
    return lax.gather(v, idx, _BCAST_DNUMS, (1,),
                      mode=lax.GatherScatterMode.PROMISE_IN_BOUNDS)


def _rot8(v):
    idx = ((lax.iota(jnp.int32, L) + 8) & 15).reshape(L, 1)
    return lax.gather(v, idx, _BCAST_DNUMS, (1,),
                      mode=lax.GatherScatterMode.PROMISE_IN_BOUNDS)


def _make_edge_sweep(R, K, H):
    RC = R // L
    RW = R + L
    NB = EPW // K
    mesh = plsc.VectorSubcoreMesh(core_axis_name="c", subcore_axis_name="s")

    slot_types = (
        pltpu.VMEM((K,), jnp.int32),
        pltpu.VMEM((K, L), jnp.float32),
        pltpu.VMEM((K, RW), jnp.float32),
        pltpu.VMEM((K, RW), jnp.float32),
        pltpu.SemaphoreType.DMA,
        pltpu.SemaphoreType.DMA,
        pltpu.SemaphoreType.DMA,
    )

    @functools.partial(
        pl.kernel,
        out_type=jax.ShapeDtypeStruct((NC, N, RW), jnp.float32),
        mesh=mesh,
        compiler_params=pltpu.CompilerParams(use_tc_tiling_on_sc=False),
        scratch_types=(
            pltpu.VMEM_SHARED((N, RW), jnp.float32),
            pltpu.VMEM((L,), jnp.float32),
            pltpu.VMEM((NB, K), jnp.int32),
        ) + slot_types + slot_types,
    )
    def sweep(src_hbm, dst2_hbm, hx_hbm, rle_hbm, m_hbm,
              out_hbm,
              out_sp, m_v, dst_all, *slot_refs):
        ns = len(slot_types)
        slots = (slot_refs[:ns], slot_refs[ns:])
        cid = lax.axis_index("c")
        tid = lax.axis_index("s")
        wid = cid * NS + tid
        rbase = tid * ROWS_T

        ebase = wid * EPW
        pltpu.sync_copy(dst2_hbm.at[pl.ds(wid * NB, NB)], dst_all)

        zmsg = slots[0][3]

        def zo(i, c):
            for j in range(RW // L):
                zmsg[i, pl.ds(j * L, L)] = jnp.zeros((L,), jnp.float32)
            return c

        lax.fori_loop(0, K, zo, 0, unroll=4)
        for z in range(ROWS_T // K):
            pltpu.sync_copy(zmsg, out_sp.at[pl.ds(rbase + z * K, K)])
        rem = ROWS_T - (ROWS_T // K) * K
        pltpu.sync_copy(zmsg.at[pl.ds(0, rem)],
                        out_sp.at[pl.ds(rbase + ROWS_T - rem, rem)])

        @pl.when(tid == NS - 1)
        def _():
            tb = NS * ROWS_T
            pltpu.sync_copy(zmsg.at[pl.ds(0, TAIL)],
                            out_sp.at[pl.ds(tb, TAIL)])

        pltpu.sync_copy(m_hbm, m_v)
        plsc.subcore_barrier()

        lane = lax.iota(jnp.int32, L)
        headmask = lane < 8
        mx = m_v[...]
        ms = mx + _rot8(mx)
        ms = jnp.where(ms > 0, ms, 0.2 * ms)
        mvec = jnp.where(lane < H, ms, 1e30)

        def prefetch_src(s, b):
            src_v, _, _, _, semi, _, _ = slots[s]
            gb = pl.multiple_of(ebase + b * K, 8)
            pltpu.async_copy(src_hbm.at[pl.ds(gb, K)], src_v, semi)

        def issue_gathers(s, b):
            src_v, rled_v, rows_v, _, semi, semg, _ = slots[s]
            gb = pl.multiple_of(ebase + b * K, 8)
            pltpu.make_async_copy(src_hbm.at[pl.ds(gb, K)], src_v, semi).wait()
            pltpu.async_copy(rle_hbm.at[dst_all.at[b]], rled_v, semg)
            pltpu.async_copy(hx_hbm.at[src_v], rows_v, semg)

        def wait_gathers(s, b):
            src_v, rled_v, rows_v, _, _, semg, _ = slots[s]
            pltpu.make_async_copy(rle_hbm.at[dst_all.at[b]],
                                  rled_v, semg).wait()
            pltpu.make_async_copy(hx_hbm.at[src_v], rows_v, semg).wait()

        def wait_scatters(s, b):
            _, _, _, msg_v, _, _, sems = slots[s]
            pltpu.make_async_copy(msg_v, out_sp.at[dst_all.at[b]], sems).wait()

        def compute_and_scatter(s, b):
            _, rled_v, rows_v, msg_v, _, _, sems = slots[s]

            @plsc.parallel_loop(0, K, unroll=4)
            def _(k):
                e = rows_v[k, pl.ds(R, L)] + rled_v[k, :]
                e = jnp.where(e > 0, e, 0.2 * e)
                wrow = jnp.where(headmask, jnp.exp(e - mvec), 0.0)
                msg_v[k, pl.ds(R, L)] = wrow
                for j in range(RC):
                    msg_v[k, pl.ds(j * L, L)] = (
                        rows_v[k, pl.ds(j * L, L)] * _lane_bcast(wrow, j))

            pltpu.async_copy(msg_v, out_sp.at[dst_all.at[b]], sems, add=True)

        NPAIRS = NB // 2

        prefetch_src(0, 0)
        prefetch_src(1, 1)
        issue_gathers(0, 0)
        issue_gathers(1, 1)

        def pair(p, c):
            wait_gathers(0, 2 * p)

            @pl.when(2 * p + 2 < NB)
            def _():
                prefetch_src(0, 2 * p + 2)

            @pl.when(p > 0)
            def _():
                wait_scatters(0, 2 * p - 2)

            compute_and_scatter(0, 2 * p)

            @pl.when(2 * p + 2 < NB)
            def _():
                issue_gathers(0, 2 * p + 2)

            wait_gathers(1, 2 * p + 1)

            @pl.when(2 * p + 3 < NB)
            def _():
                prefetch_src(1, 2 * p + 3)

            @pl.when(p > 0)
            def _():
                wait_scatters(1, 2 * p - 1)

            compute_and_scatter(1, 2 * p + 1)

            @pl.when(2 * p + 3 < NB)
            def _():
                issue_gathers(1, 2 * p + 3)

            return c

        lax.fori_loop(0, NPAIRS, pair, 0)

        if NB % 2:
            wait_gathers(0, NB - 1)
            wait_scatters(0, NB - 3)
            compute_and_scatter(0, NB - 1)
            wait_scatters(0, NB - 1)
            wait_scatters(1, NB - 2)
        else:
            wait_scatters(0, NB - 2)
            wait_scatters(1, NB - 1)

        plsc.subcore_barrier()
        pltpu.sync_copy(out_sp.at[pl.ds(rbase, ROWS_T)],
                        out_hbm.at[cid, pl.ds(rbase, ROWS_T)])

        @pl.when(tid == NS - 1)
        def _():
            tb = NS * ROWS_T
            pltpu.sync_copy(out_sp.at[pl.ds(tb, TAIL)],
                            out_hbm.at[cid, pl.ds(tb, TAIL)])

    return sweep


_edge_sweep_128 = _make_edge_sweep(D, K1, 8)
_edge_sweep_16 = _make_edge_sweep(L, K2, 1)


def _pack(al, ar):
    H, Fo = al.shape
    eye = jnp.eye(H, 8, dtype=jnp.float32)
    a_el = (al[:, :, None] * eye[:, None, :]).reshape(H * Fo, 8)
    a_er = (ar[:, :, None] * eye[:, None, :]).reshape(H * Fo, 8)
    return (jnp.concatenate([a_el, a_er], 1).astype(jnp.float32),
            jnp.concatenate([a_er, a_el], 1).astype(jnp.float32))


def kernel(feats, edge_index, W1, al1, ar1, b1, W2, al2, ar2, b2):
    pe1, pr1 = _pack(al1, ar1)
    pe2, pr2 = _pack(al2, ar2)
    q = (jnp.eye(L, 8, dtype=jnp.float32)[:, :, None]
         * jnp.ones((1, 1, L), jnp.float32)).reshape(L, D)
    q2 = jnp.zeros((L, L), jnp.float32).at[0, :].set(1.0)

    src = edge_index[0]
    dst = edge_index[1]
    hx1, rle1, mx1 = _tc_project(feats, W1, pe1, pr1)
    out1p = _edge_sweep_128(src, dst.reshape(E // K1, K1), hx1, rle1,
                            mx1.reshape(L))

    hx2, rle2, mx2 = _tc_combine_project(
        out1p, W2, q, b1.reshape(1, D), pe2, pr2)
    out2p = _edge_sweep_16(src, dst.reshape(E // K2, K2), hx2, rle2,
                           mx2.reshape(L))

    return _tc_finish(out2p, q2, b2.reshape(1, L))
